# Initial kernel scaffold; baseline (speedup 1.0000x reference)
import functools

import jax
import jax.numpy as jnp
from jax import lax
from jax.experimental import pallas as pl
from jax.experimental.pallas import tpu as pltpu
from jax.experimental.pallas import tpu_sc as plsc

N_NODES = 50000
N_EDGES = 800000
ALPHA = 0.01
EPS = 1e-5

NP = 50176
ZR = NP // 16
EPAD = 802816
ROWS = EPAD // 128
ROWS_H = ROWS // 2
GB = 7
RBLK = 1024
BN_S = float(1.0 / (1.0 + EPS) ** 0.5)


def _sc_body(t0, t1, src_hbm, dst_hbm, zeros_hbm, out0, out1,
             accum, srcb, dstb, rbuf, gsem, ssem, *, off0, cnt0, off1, cnt1):
    c = lax.axis_index("c")
    s = lax.axis_index("s")

    pltpu.sync_copy(zeros_hbm.at[pl.ds(s * ZR, ZR)], accum.at[pl.ds(s * ZR, ZR)])
    plsc.subcore_barrier()

    def do_pass(table, off, cnt):
        per_tile = cnt // 16
        base = off + s * per_tile

        def it(t, carry):
            r0 = base + t * GB
            pltpu.sync_copy(src_hbm.at[pl.ds(r0, GB)], srcb)
            pltpu.sync_copy(dst_hbm.at[pl.ds(r0, GB)], dstb)
            hs = [pltpu.async_copy(table.at[srcb.at[b]], rbuf.at[b], gsem)
                  for b in range(GB)]
            for h in hs:
                h.wait()
            ss = [pltpu.async_copy(rbuf.at[b], accum.at[dstb.at[b]], ssem,
                                   add=True)
                  for b in range(GB)]
            for h in ss:
                h.wait()
            return carry

        lax.fori_loop(0, per_tile // GB, it, 0)

    @pl.when(c == 0)
    def _():
        do_pass(t0, off0, cnt0)

    @pl.when(c == 1)
    def _():
        do_pass(t1, off1, cnt1)

    plsc.subcore_barrier()

    @pl.when(c == 0)
    def _():
        pltpu.sync_copy(accum.at[pl.ds(s * ZR, ZR)], out0.at[pl.ds(s * ZR, ZR)])

    @pl.when(c == 1)
    def _():
        pltpu.sync_copy(accum.at[pl.ds(s * ZR, ZR)], out1.at[pl.ds(s * ZR, ZR)])


def _sc_pass(t0, t1, src2d, dst2d, zeros, *, off0, cnt0, off1, cnt1):
    d = t0.shape[1]
    f32 = jnp.float32
    body = functools.partial(_sc_body, off0=off0, cnt0=cnt0, off1=off1,
                             cnt1=cnt1)
    return pl.kernel(
        body,
        out_type=(jax.ShapeDtypeStruct((NP, d), f32),
                  jax.ShapeDtypeStruct((NP, d), f32)),
        mesh=plsc.VectorSubcoreMesh(core_axis_name="c", subcore_axis_name="s"),
        scratch_types=(
            pltpu.VMEM_SHARED((NP, d), f32),
            pltpu.VMEM((GB, 128), jnp.int32),
            pltpu.VMEM((GB, 128), jnp.int32),
            pltpu.VMEM((GB, 128, d), f32),
            pltpu.SemaphoreType.DMA,
            pltpu.SemaphoreType.DMA,
        ),
        name=f"gcn_sc_scatter_d{d}_{cnt0}",
    )(t0, t1, src2d, dst2d, zeros)


def _t1_body(emb, dega, degb, xs_out, dinv_out):
    d8 = dega[...] + degb[...] + 1.0
    dinv8 = lax.rsqrt(d8)
    dinv_out[...] = dinv8
    xs_out[...] = emb[...] * dinv8[:, 0:1]


def _t2_body(p1a, p1b, xs, dinv8, W1, b1, g1, be1, lo_out, hi_out):
    dinv = dinv8[:, 0:1]
    P = (p1a[...] + p1b[...] + xs[...]) * dinv
    z = jnp.dot(P, W1[...], preferred_element_type=jnp.float32)
    A = g1[...] * BN_S
    B = b1[...] * A + be1[...]
    y = z * A + B
    y = jnp.where(y >= 0, y, ALPHA * y)
    ys = y * dinv
    lo_out[...] = ys[:, :32]
    hi_out[...] = ys[:, 32:]


def _t3_body(p2lo, p2hi, yslo, yshi, dinv8, W2, b2, g2, be2, Wf, bf, out):
    dinv = dinv8[:, 0:1]
    Plo = (p2lo[...] + yslo[...]) * dinv
    Phi = (p2hi[...] + yshi[...]) * dinv
    z = (jnp.dot(Plo, W2[:32, :], preferred_element_type=jnp.float32)
         + jnp.dot(Phi, W2[32:, :], preferred_element_type=jnp.float32))
    A = g2[...] * BN_S
    B = b2[...] * A + be2[...]
    y = z * A + B
    y = jnp.where(y >= 0, y, ALPHA * y)
    out[...] = jnp.dot(y, Wf[...], preferred_element_type=jnp.float32) + bf[...]


def _row_spec(w):
    return pl.BlockSpec((RBLK, w), lambda i: (i, 0))


def _full_spec(shape):
    nd = len(shape)
    return pl.BlockSpec(shape, lambda i: (0,) * nd)


def kernel(embeddings, edge_index, W1, b1, g1, be1, W2, b2, g2, be2, Wf, bf):
    f32 = jnp.float32
    grid = (NP // RBLK,)

    emb_pad = jnp.pad(embeddings, ((0, NP - N_NODES), (0, 0)))
    src = edge_index[0].astype(jnp.int32)
    dst = edge_index[1].astype(jnp.int32)
    pad_idx = jnp.full((EPAD - N_EDGES,), N_NODES, jnp.int32)
    src2d = jnp.concatenate([src, pad_idx]).reshape(ROWS, 128)
    dst2d = jnp.concatenate([dst, pad_idx]).reshape(ROWS, 128)
    zeros8 = jnp.zeros((NP, 8), f32)
    zeros32 = jnp.zeros((NP, 32), f32)
    ones8 = zeros8.at[:N_NODES, 0].set(1.0)

    dega, degb = _sc_pass(ones8, ones8, src2d, dst2d, zeros8,
                          off0=0, cnt0=ROWS_H, off1=ROWS_H, cnt1=ROWS_H)

    xs, dinv8 = pl.pallas_call(
        _t1_body,
        grid=grid,
        in_specs=[_row_spec(32), _row_spec(8), _row_spec(8)],
        out_specs=[_row_spec(32), _row_spec(8)],
        out_shape=[jax.ShapeDtypeStruct((NP, 32), f32),
                   jax.ShapeDtypeStruct((NP, 8), f32)],
        name="gcn_tc_prep",
    )(emb_pad, dega, degb)

    p1a, p1b = _sc_pass(xs, xs, src2d, dst2d, zeros32,
                        off0=0, cnt0=ROWS_H, off1=ROWS_H, cnt1=ROWS_H)

    ys_lo, ys_hi = pl.pallas_call(
        _t2_body,
        grid=grid,
        in_specs=[_row_spec(32), _row_spec(32), _row_spec(32), _row_spec(8),
                  _full_spec((32, 64)), _full_spec((1, 64)),
                  _full_spec((1, 64)), _full_spec((1, 64))],
        out_specs=[_row_spec(32), _row_spec(32)],
        out_shape=[jax.ShapeDtypeStruct((NP, 32), f32),
                   jax.ShapeDtypeStruct((NP, 32), f32)],
        name="gcn_tc_layer1",
    )(p1a, p1b, xs, dinv8, W1, b1.reshape(1, -1), g1.reshape(1, -1),
      be1.reshape(1, -1))

    p2lo, p2hi = _sc_pass(ys_lo, ys_hi, src2d, dst2d, zeros32,
                          off0=0, cnt0=ROWS, off1=0, cnt1=ROWS)

    out = pl.pallas_call(
        _t3_body,
        grid=grid,
        in_specs=[_row_spec(32), _row_spec(32), _row_spec(32), _row_spec(32),
                  _row_spec(8),
                  _full_spec((64, 128)), _full_spec((1, 128)),
                  _full_spec((1, 128)), _full_spec((1, 128)),
                  _full_spec((128, 128)), _full_spec((1, 128))],
        out_specs=_row_spec(128),
        out_shape=jax.ShapeDtypeStruct((NP, 128), f32),
        name="gcn_tc_layer2_final",
    )(p2lo, p2hi, ys_lo, ys_hi, dinv8, W2, b2.reshape(1, -1),
      g2.reshape(1, -1), be2.reshape(1, -1), Wf, bf.reshape(1, -1))

    return out[:N_NODES]


# speedup vs baseline: 16.4643x; 16.4643x over previous
import functools

import jax
import jax.numpy as jnp
from jax import lax
from jax.experimental import pallas as pl
from jax.experimental.pallas import tpu as pltpu
from jax.experimental.pallas import tpu_sc as plsc

N_NODES = 50000
N_EDGES = 800000
ALPHA = 0.01
EPS = 1e-5

NP = 50176
ZR = NP // 16
EPAD = 819200
ROWS = EPAD // 128
ROWS_H = ROWS // 2
GB = 8
NBUF = 4
RBLK = 1024
BN_S = float(1.0 / (1.0 + EPS) ** 0.5)


def _sc_body(t0, t1, src_hbm, dst_hbm, zeros_hbm, out0, out1,
             accum, srcb, dstb, rbuf, gsem, ssem, *, off0, cnt0, off1, cnt1):
    c = lax.axis_index("c")
    s = lax.axis_index("s")

    pltpu.sync_copy(zeros_hbm.at[pl.ds(s * ZR, ZR)], accum.at[pl.ds(s * ZR, ZR)])
    plsc.subcore_barrier()

    def do_pass(table, off, cnt):
        per_tile = cnt // 16
        base = off + s * per_tile

        def it(t, carry):
            r0 = base + t * GB
            pltpu.sync_copy(src_hbm.at[pl.ds(r0, GB)], srcb)
            pltpu.sync_copy(dst_hbm.at[pl.ds(r0, GB)], dstb)
            for h0 in range(GB // NBUF):
                hs = [pltpu.async_copy(table.at[srcb.at[h0 * NBUF + b]],
                                       rbuf.at[b], gsem)
                      for b in range(NBUF)]
                for h in hs:
                    h.wait()
                ss = [pltpu.async_copy(rbuf.at[b],
                                       accum.at[dstb.at[h0 * NBUF + b]],
                                       ssem, add=True)
                      for b in range(NBUF)]
                for h in ss:
                    h.wait()
            return carry

        lax.fori_loop(0, per_tile // GB, it, 0)

    @pl.when(c == 0)
    def _():
        do_pass(t0, off0, cnt0)

    @pl.when(c == 1)
    def _():
        do_pass(t1, off1, cnt1)

    plsc.subcore_barrier()

    @pl.when(c == 0)
    def _():
        pltpu.sync_copy(accum.at[pl.ds(s * ZR, ZR)], out0.at[pl.ds(s * ZR, ZR)])

    @pl.when(c == 1)
    def _():
        pltpu.sync_copy(accum.at[pl.ds(s * ZR, ZR)], out1.at[pl.ds(s * ZR, ZR)])


def _sc_pass(t0, t1, src2d, dst2d, zeros, *, off0, cnt0, off1, cnt1):
    d = t0.shape[1]
    f32 = jnp.float32
    body = functools.partial(_sc_body, off0=off0, cnt0=cnt0, off1=off1,
                             cnt1=cnt1)
    return pl.kernel(
        body,
        out_type=(jax.ShapeDtypeStruct((NP, d), f32),
                  jax.ShapeDtypeStruct((NP, d), f32)),
        mesh=plsc.VectorSubcoreMesh(core_axis_name="c", subcore_axis_name="s"),
        scratch_types=(
            pltpu.VMEM_SHARED((NP, d), f32),
            pltpu.VMEM((GB, 128), jnp.int32),
            pltpu.VMEM((GB, 128), jnp.int32),
            pltpu.VMEM((NBUF, 128, d), f32),
            pltpu.SemaphoreType.DMA,
            pltpu.SemaphoreType.DMA,
        ),
        compiler_params=pltpu.CompilerParams(use_tc_tiling_on_sc=False),
        name=f"gcn_sc_scatter_d{d}_{cnt0}",
    )(t0, t1, src2d, dst2d, zeros)


def _t1_body(emb, dega, degb, xs_out, dinv_out):
    d8 = dega[...] + degb[...] + 1.0
    dinv8 = lax.rsqrt(d8)
    dinv_out[...] = dinv8
    xs_out[...] = emb[...] * dinv8[:, 0:1]


def _t2_body(p1a, p1b, xs, dinv8, W1, b1, g1, be1, lo_out, hi_out):
    dinv = dinv8[:, 0:1]
    P = (p1a[...] + p1b[...] + xs[...]) * dinv
    z = jnp.dot(P, W1[...], preferred_element_type=jnp.float32)
    A = g1[...] * BN_S
    B = b1[...] * A + be1[...]
    y = z * A + B
    y = jnp.where(y >= 0, y, ALPHA * y)
    ys = y * dinv
    lo_out[...] = ys[:, :32]
    hi_out[...] = ys[:, 32:]


def _t3_body(p2lo, p2hi, yslo, yshi, dinv8, W2, b2, g2, be2, Wf, bf, out):
    dinv = dinv8[:, 0:1]
    Plo = (p2lo[...] + yslo[...]) * dinv
    Phi = (p2hi[...] + yshi[...]) * dinv
    z = (jnp.dot(Plo, W2[:32, :], preferred_element_type=jnp.float32)
         + jnp.dot(Phi, W2[32:, :], preferred_element_type=jnp.float32))
    A = g2[...] * BN_S
    B = b2[...] * A + be2[...]
    y = z * A + B
    y = jnp.where(y >= 0, y, ALPHA * y)
    out[...] = jnp.dot(y, Wf[...], preferred_element_type=jnp.float32) + bf[...]


def _row_spec(w):
    return pl.BlockSpec((RBLK, w), lambda i: (i, 0))


def _full_spec(shape):
    nd = len(shape)
    return pl.BlockSpec(shape, lambda i: (0,) * nd)


def kernel(embeddings, edge_index, W1, b1, g1, be1, W2, b2, g2, be2, Wf, bf):
    f32 = jnp.float32
    grid = (NP // RBLK,)

    emb_pad = jnp.pad(embeddings, ((0, NP - N_NODES), (0, 0)))
    src = edge_index[0].astype(jnp.int32)
    dst = edge_index[1].astype(jnp.int32)
    pad_idx = jnp.full((EPAD - N_EDGES,), N_NODES, jnp.int32)
    src2d = jnp.concatenate([src, pad_idx]).reshape(ROWS, 128)
    dst2d = jnp.concatenate([dst, pad_idx]).reshape(ROWS, 128)
    zeros8 = jnp.zeros((NP, 8), f32)
    zeros32 = jnp.zeros((NP, 32), f32)
    ones8 = zeros8.at[:N_NODES, 0].set(1.0)

    dega, degb = _sc_pass(ones8, ones8, src2d, dst2d, zeros8,
                          off0=0, cnt0=ROWS_H, off1=ROWS_H, cnt1=ROWS_H)

    xs, dinv8 = pl.pallas_call(
        _t1_body,
        grid=grid,
        in_specs=[_row_spec(32), _row_spec(8), _row_spec(8)],
        out_specs=[_row_spec(32), _row_spec(8)],
        out_shape=[jax.ShapeDtypeStruct((NP, 32), f32),
                   jax.ShapeDtypeStruct((NP, 8), f32)],
        name="gcn_tc_prep",
    )(emb_pad, dega, degb)

    p1a, p1b = _sc_pass(xs, xs, src2d, dst2d, zeros32,
                        off0=0, cnt0=ROWS_H, off1=ROWS_H, cnt1=ROWS_H)

    ys_lo, ys_hi = pl.pallas_call(
        _t2_body,
        grid=grid,
        in_specs=[_row_spec(32), _row_spec(32), _row_spec(32), _row_spec(8),
                  _full_spec((32, 64)), _full_spec((1, 64)),
                  _full_spec((1, 64)), _full_spec((1, 64))],
        out_specs=[_row_spec(32), _row_spec(32)],
        out_shape=[jax.ShapeDtypeStruct((NP, 32), f32),
                   jax.ShapeDtypeStruct((NP, 32), f32)],
        name="gcn_tc_layer1",
    )(p1a, p1b, xs, dinv8, W1, b1.reshape(1, -1), g1.reshape(1, -1),
      be1.reshape(1, -1))

    p2lo, p2hi = _sc_pass(ys_lo, ys_hi, src2d, dst2d, zeros32,
                          off0=0, cnt0=ROWS, off1=0, cnt1=ROWS)

    out = pl.pallas_call(
        _t3_body,
        grid=grid,
        in_specs=[_row_spec(32), _row_spec(32), _row_spec(32), _row_spec(32),
                  _row_spec(8),
                  _full_spec((64, 128)), _full_spec((1, 128)),
                  _full_spec((1, 128)), _full_spec((1, 128)),
                  _full_spec((128, 128)), _full_spec((1, 128))],
        out_specs=_row_spec(128),
        out_shape=jax.ShapeDtypeStruct((NP, 128), f32),
        name="gcn_tc_layer2_final",
    )(p2lo, p2hi, ys_lo, ys_hi, dinv8, W2, b2.reshape(1, -1),
      g2.reshape(1, -1), be2.reshape(1, -1), Wf, bf.reshape(1, -1))

    return out[:N_NODES]


# speedup vs baseline: 18.6772x; 1.1344x over previous
import functools

import jax
import jax.numpy as jnp
from jax import lax
from jax.experimental import pallas as pl
from jax.experimental.pallas import tpu as pltpu
from jax.experimental.pallas import tpu_sc as plsc

N_NODES = 50000
N_EDGES = 800000
ALPHA = 0.01
EPS = 1e-5

NP = 50176
ZR = NP // 16
EPAD = 819200
ROWS = EPAD // 128
ROWS_H = ROWS // 2
GB = 8
NBUF = 4
RBLK = 1024
BN_S = float(1.0 / (1.0 + EPS) ** 0.5)


def _sc_body(t0, t1, src_hbm, dst_hbm, zeros_hbm, out0, out1,
             accum, srcb, dstb, rbuf, isem, gsem, ssem,
             *, off0, cnt0, off1, cnt1):
    c = lax.axis_index("c")
    s = lax.axis_index("s")

    pltpu.sync_copy(zeros_hbm.at[pl.ds(s * ZR, ZR)], accum.at[pl.ds(s * ZR, ZR)])
    plsc.subcore_barrier()

    def do_pass(table, off, cnt):
        per_tile = cnt // 16
        base = off + s * per_tile
        nsc = per_tile // GB

        def load_idx(g, sl):
            pltpu.async_copy(src_hbm.at[pl.ds(base + g * GB, GB)],
                             srcb.at[sl], isem)
            pltpu.async_copy(dst_hbm.at[pl.ds(base + g * GB, GB)],
                             dstb.at[sl], isem)

        def wait_idx(sl):
            for ib in (srcb, dstb):
                pltpu.make_async_copy(src_hbm.at[pl.ds(base, GB)],
                                      ib.at[sl], isem).wait()

        def drain(sem, slot):
            pltpu.make_async_copy(zeros_hbm.at[pl.ds(0, 128)],
                                  rbuf.at[slot], sem).wait()

        def gather(sl, b, slot):
            pltpu.async_copy(table.at[srcb.at[sl].at[b]], rbuf.at[slot], gsem)

        def scatter(sl, b, slot):
            pltpu.async_copy(rbuf.at[slot], accum.at[dstb.at[sl].at[b]],
                             ssem, add=True)

        load_idx(0, 0)

        def g_body(g, carry):
            sl = lax.rem(g, 2)

            @pl.when(g > 0)
            def _():
                drain(gsem, 3)
                scatter(1 - sl, GB - 1, 3)

            wait_idx(sl)

            for b in range(GB):
                slot = b % 4
                if b < 4:
                    @pl.when(g > 0)
                    def _():
                        drain(ssem, slot)
                else:
                    drain(ssem, slot)
                if b == 4:
                    @pl.when(g + 1 < nsc)
                    def _():
                        load_idx(g + 1, 1 - sl)
                gather(sl, b, slot)
                if b > 0:
                    drain(gsem, (b - 1) % 4)
                    scatter(sl, b - 1, (b - 1) % 4)
            return carry

        lax.fori_loop(0, nsc, g_body, 0)

        drain(gsem, 3)
        scatter((nsc - 1) % 2, GB - 1, 3)
        for k in range(4):
            drain(ssem, k)

    @pl.when(c == 0)
    def _():
        do_pass(t0, off0, cnt0)

    @pl.when(c == 1)
    def _():
        do_pass(t1, off1, cnt1)

    plsc.subcore_barrier()

    @pl.when(c == 0)
    def _():
        pltpu.sync_copy(accum.at[pl.ds(s * ZR, ZR)], out0.at[pl.ds(s * ZR, ZR)])

    @pl.when(c == 1)
    def _():
        pltpu.sync_copy(accum.at[pl.ds(s * ZR, ZR)], out1.at[pl.ds(s * ZR, ZR)])


def _sc_pass(t0, t1, src2d, dst2d, zeros, *, off0, cnt0, off1, cnt1):
    d = t0.shape[1]
    f32 = jnp.float32
    body = functools.partial(_sc_body, off0=off0, cnt0=cnt0, off1=off1,
                             cnt1=cnt1)
    return pl.kernel(
        body,
        out_type=(jax.ShapeDtypeStruct((NP, d), f32),
                  jax.ShapeDtypeStruct((NP, d), f32)),
        mesh=plsc.VectorSubcoreMesh(core_axis_name="c", subcore_axis_name="s"),
        scratch_types=(
            pltpu.VMEM_SHARED((NP, d), f32),
            pltpu.VMEM((2, GB, 128), jnp.int32),
            pltpu.VMEM((2, GB, 128), jnp.int32),
            pltpu.VMEM((NBUF, 128, d), f32),
            pltpu.SemaphoreType.DMA,
            pltpu.SemaphoreType.DMA,
            pltpu.SemaphoreType.DMA,
        ),
        compiler_params=pltpu.CompilerParams(use_tc_tiling_on_sc=False),
        name=f"gcn_sc_scatter_d{d}_{cnt0}",
    )(t0, t1, src2d, dst2d, zeros)


def _t1_body(emb, dega, degb, xs_out, dinv_out):
    d8 = dega[...] + degb[...] + 1.0
    dinv8 = lax.rsqrt(d8)
    dinv_out[...] = dinv8
    xs_out[...] = emb[...] * dinv8[:, 0:1]


def _t2_body(p1a, p1b, xs, dinv8, W1, b1, g1, be1, lo_out, hi_out):
    dinv = dinv8[:, 0:1]
    P = (p1a[...] + p1b[...] + xs[...]) * dinv
    z = jnp.dot(P, W1[...], preferred_element_type=jnp.float32)
    A = g1[...] * BN_S
    B = b1[...] * A + be1[...]
    y = z * A + B
    y = jnp.where(y >= 0, y, ALPHA * y)
    ys = y * dinv
    lo_out[...] = ys[:, :32]
    hi_out[...] = ys[:, 32:]


def _t3_body(p2lo, p2hi, yslo, yshi, dinv8, W2, b2, g2, be2, Wf, bf, out):
    dinv = dinv8[:, 0:1]
    Plo = (p2lo[...] + yslo[...]) * dinv
    Phi = (p2hi[...] + yshi[...]) * dinv
    z = (jnp.dot(Plo, W2[:32, :], preferred_element_type=jnp.float32)
         + jnp.dot(Phi, W2[32:, :], preferred_element_type=jnp.float32))
    A = g2[...] * BN_S
    B = b2[...] * A + be2[...]
    y = z * A + B
    y = jnp.where(y >= 0, y, ALPHA * y)
    out[...] = jnp.dot(y, Wf[...], preferred_element_type=jnp.float32) + bf[...]


def _row_spec(w):
    return pl.BlockSpec((RBLK, w), lambda i: (i, 0))


def _full_spec(shape):
    nd = len(shape)
    return pl.BlockSpec(shape, lambda i: (0,) * nd)


def kernel(embeddings, edge_index, W1, b1, g1, be1, W2, b2, g2, be2, Wf, bf):
    f32 = jnp.float32
    grid = (NP // RBLK,)

    emb_pad = jnp.pad(embeddings, ((0, NP - N_NODES), (0, 0)))
    src = edge_index[0].astype(jnp.int32)
    dst = edge_index[1].astype(jnp.int32)
    pad_idx = jnp.full((EPAD - N_EDGES,), N_NODES, jnp.int32)
    src2d = jnp.concatenate([src, pad_idx]).reshape(ROWS, 128)
    dst2d = jnp.concatenate([dst, pad_idx]).reshape(ROWS, 128)
    zeros8 = jnp.zeros((NP, 8), f32)
    zeros32 = jnp.zeros((NP, 32), f32)
    ones8 = zeros8.at[:N_NODES, 0].set(1.0)

    dega, degb = _sc_pass(ones8, ones8, src2d, dst2d, zeros8,
                          off0=0, cnt0=ROWS_H, off1=ROWS_H, cnt1=ROWS_H)

    xs, dinv8 = pl.pallas_call(
        _t1_body,
        grid=grid,
        in_specs=[_row_spec(32), _row_spec(8), _row_spec(8)],
        out_specs=[_row_spec(32), _row_spec(8)],
        out_shape=[jax.ShapeDtypeStruct((NP, 32), f32),
                   jax.ShapeDtypeStruct((NP, 8), f32)],
        name="gcn_tc_prep",
    )(emb_pad, dega, degb)

    p1a, p1b = _sc_pass(xs, xs, src2d, dst2d, zeros32,
                        off0=0, cnt0=ROWS_H, off1=ROWS_H, cnt1=ROWS_H)

    ys_lo, ys_hi = pl.pallas_call(
        _t2_body,
        grid=grid,
        in_specs=[_row_spec(32), _row_spec(32), _row_spec(32), _row_spec(8),
                  _full_spec((32, 64)), _full_spec((1, 64)),
                  _full_spec((1, 64)), _full_spec((1, 64))],
        out_specs=[_row_spec(32), _row_spec(32)],
        out_shape=[jax.ShapeDtypeStruct((NP, 32), f32),
                   jax.ShapeDtypeStruct((NP, 32), f32)],
        name="gcn_tc_layer1",
    )(p1a, p1b, xs, dinv8, W1, b1.reshape(1, -1), g1.reshape(1, -1),
      be1.reshape(1, -1))

    p2lo, p2hi = _sc_pass(ys_lo, ys_hi, src2d, dst2d, zeros32,
                          off0=0, cnt0=ROWS, off1=0, cnt1=ROWS)

    out = pl.pallas_call(
        _t3_body,
        grid=grid,
        in_specs=[_row_spec(32), _row_spec(32), _row_spec(32), _row_spec(32),
                  _row_spec(8),
                  _full_spec((64, 128)), _full_spec((1, 128)),
                  _full_spec((1, 128)), _full_spec((1, 128)),
                  _full_spec((128, 128)), _full_spec((1, 128))],
        out_specs=_row_spec(128),
        out_shape=jax.ShapeDtypeStruct((NP, 128), f32),
        name="gcn_tc_layer2_final",
    )(p2lo, p2hi, ys_lo, ys_hi, dinv8, W2, b2.reshape(1, -1),
      g2.reshape(1, -1), be2.reshape(1, -1), Wf, bf.reshape(1, -1))

    return out[:N_NODES]


# speedup vs baseline: 29.0502x; 1.5554x over previous
import functools

import jax
import jax.numpy as jnp
from jax import lax
from jax.experimental import pallas as pl
from jax.experimental.pallas import tpu as pltpu
from jax.experimental.pallas import tpu_sc as plsc

N_NODES = 50000
N_EDGES = 800000
ALPHA = 0.01
EPS = 1e-5

NP = 50176
ZR = NP // 16
EPAD = 819200
ROWS = EPAD // 128
ROWS_H = ROWS // 2
GB = 8
NBUF = 4
RBLK = 1024
BN_S = float(1.0 / (1.0 + EPS) ** 0.5)


def _sc_body(t0, t1, src_hbm, dst_hbm, zeros_hbm, out0, out1,
             accum, srcb, dstb, rbuf, isem, gsem, ssem,
             *, off0, cnt0, off1, cnt1):
    c = lax.axis_index("c")
    s = lax.axis_index("s")

    pltpu.sync_copy(zeros_hbm.at[pl.ds(s * ZR, ZR)], accum.at[pl.ds(s * ZR, ZR)])
    plsc.subcore_barrier()

    def do_pass(table, off, cnt):
        per_tile = cnt // 16
        base = off + s * per_tile
        nsc = per_tile // GB

        def load_idx(g, sl):
            pltpu.async_copy(src_hbm.at[pl.ds(base + g * GB, GB)],
                             srcb.at[sl], isem)
            pltpu.async_copy(dst_hbm.at[pl.ds(base + g * GB, GB)],
                             dstb.at[sl], isem)

        def wait_idx(sl):
            for ib in (srcb, dstb):
                pltpu.make_async_copy(src_hbm.at[pl.ds(base, GB)],
                                      ib.at[sl], isem).wait()

        def drain(sem, slot):
            pltpu.make_async_copy(zeros_hbm.at[pl.ds(0, 128)],
                                  rbuf.at[slot], sem).wait()

        def gather(sl, b, slot):
            pltpu.async_copy(table.at[srcb.at[sl].at[b]], rbuf.at[slot], gsem)

        def scatter(sl, b, slot):
            pltpu.async_copy(rbuf.at[slot], accum.at[dstb.at[sl].at[b]],
                             ssem, add=True)

        load_idx(0, 0)

        def g_body(g, carry):
            sl = lax.rem(g, 2)

            @pl.when(g > 0)
            def _():
                drain(gsem, 3)
                scatter(1 - sl, GB - 1, 3)

            wait_idx(sl)

            for b in range(GB):
                slot = b % 4
                if b < 4:
                    @pl.when(g > 0)
                    def _():
                        drain(ssem, slot)
                else:
                    drain(ssem, slot)
                if b == 4:
                    @pl.when(g + 1 < nsc)
                    def _():
                        load_idx(g + 1, 1 - sl)
                gather(sl, b, slot)
                if b > 0:
                    drain(gsem, (b - 1) % 4)
                    scatter(sl, b - 1, (b - 1) % 4)
            return carry

        lax.fori_loop(0, nsc, g_body, 0)

        drain(gsem, 3)
        scatter((nsc - 1) % 2, GB - 1, 3)
        for k in range(4):
            drain(ssem, k)

    @pl.when(c == 0)
    def _():
        do_pass(t0, off0, cnt0)

    @pl.when(c == 1)
    def _():
        do_pass(t1, off1, cnt1)

    plsc.subcore_barrier()

    @pl.when(c == 0)
    def _():
        pltpu.sync_copy(accum.at[pl.ds(s * ZR, ZR)], out0.at[pl.ds(s * ZR, ZR)])

    @pl.when(c == 1)
    def _():
        pltpu.sync_copy(accum.at[pl.ds(s * ZR, ZR)], out1.at[pl.ds(s * ZR, ZR)])


def _sc_pass(t0, t1, src2d, dst2d, zeros, *, off0, cnt0, off1, cnt1):
    d = t0.shape[1]
    f32 = jnp.float32
    body = functools.partial(_sc_body, off0=off0, cnt0=cnt0, off1=off1,
                             cnt1=cnt1)
    return pl.kernel(
        body,
        out_type=(jax.ShapeDtypeStruct((NP, d), f32),
                  jax.ShapeDtypeStruct((NP, d), f32)),
        mesh=plsc.VectorSubcoreMesh(core_axis_name="c", subcore_axis_name="s"),
        scratch_types=(
            pltpu.VMEM_SHARED((NP, d), f32),
            pltpu.VMEM((2, GB, 128), jnp.int32),
            pltpu.VMEM((2, GB, 128), jnp.int32),
            pltpu.VMEM((NBUF, 128, d), f32),
            pltpu.SemaphoreType.DMA,
            pltpu.SemaphoreType.DMA,
            pltpu.SemaphoreType.DMA,
        ),
        compiler_params=pltpu.CompilerParams(use_tc_tiling_on_sc=False),
        name=f"gcn_sc_scatter_d{d}_{cnt0}",
    )(t0, t1, src2d, dst2d, zeros)


def _t1_body(emb, dega, degb, xs_out, dinv_out):
    d8 = dega[...] + degb[...] + 1.0
    dinv8 = lax.rsqrt(d8)
    dinv_out[...] = dinv8
    xs_out[...] = emb[...] * dinv8[:, 0:1]


def _t2_body(p1a, p1b, xs, dinv8, W1, b1, g1, be1, lo_out, hi_out):
    dinv = dinv8[:, 0:1]
    P = (p1a[...] + p1b[...] + xs[...]) * dinv
    z = jnp.dot(P, W1[...], preferred_element_type=jnp.float32)
    A = g1[...] * BN_S
    B = b1[...] * A + be1[...]
    y = z * A + B
    y = jnp.where(y >= 0, y, ALPHA * y)
    ys = y * dinv
    lo_out[...] = ys[:, :32]
    hi_out[...] = ys[:, 32:]


def _t3_body(p2lo, p2hi, yslo, yshi, dinv8, W2, b2, g2, be2, Wf, bf, out):
    dinv = dinv8[:, 0:1]
    Plo = (p2lo[...] + yslo[...]) * dinv
    Phi = (p2hi[...] + yshi[...]) * dinv
    z = (jnp.dot(Plo, W2[:32, :], preferred_element_type=jnp.float32)
         + jnp.dot(Phi, W2[32:, :], preferred_element_type=jnp.float32))
    A = g2[...] * BN_S
    B = b2[...] * A + be2[...]
    y = z * A + B
    y = jnp.where(y >= 0, y, ALPHA * y)
    out[...] = jnp.dot(y, Wf[...], preferred_element_type=jnp.float32) + bf[...]


def _row_spec(w):
    return pl.BlockSpec((RBLK, w), lambda i: (i, 0))


def _full_spec(shape):
    nd = len(shape)
    return pl.BlockSpec(shape, lambda i: (0,) * nd)


def kernel(embeddings, edge_index, W1, b1, g1, be1, W2, b2, g2, be2, Wf, bf):
    f32 = jnp.float32
    grid = (NP // RBLK,)

    emb_pad = jnp.pad(embeddings, ((0, NP - N_NODES), (0, 0)))
    src = edge_index[0].astype(jnp.int32)
    dst = edge_index[1].astype(jnp.int32)
    pad_idx = N_NODES + jnp.arange(EPAD - N_EDGES, dtype=jnp.int32) % (NP - N_NODES)
    src2d = jnp.concatenate([src, pad_idx]).reshape(ROWS, 128)
    dst2d = jnp.concatenate([dst, pad_idx]).reshape(ROWS, 128)
    zeros8 = jnp.zeros((NP, 8), f32)
    zeros32 = jnp.zeros((NP, 32), f32)
    ones8 = zeros8.at[:N_NODES, 0].set(1.0)

    dega, degb = _sc_pass(ones8, ones8, src2d, dst2d, zeros8,
                          off0=0, cnt0=ROWS_H, off1=ROWS_H, cnt1=ROWS_H)

    xs, dinv8 = pl.pallas_call(
        _t1_body,
        grid=grid,
        in_specs=[_row_spec(32), _row_spec(8), _row_spec(8)],
        out_specs=[_row_spec(32), _row_spec(8)],
        out_shape=[jax.ShapeDtypeStruct((NP, 32), f32),
                   jax.ShapeDtypeStruct((NP, 8), f32)],
        name="gcn_tc_prep",
    )(emb_pad, dega, degb)

    p1a, p1b = _sc_pass(xs, xs, src2d, dst2d, zeros32,
                        off0=0, cnt0=ROWS_H, off1=ROWS_H, cnt1=ROWS_H)

    ys_lo, ys_hi = pl.pallas_call(
        _t2_body,
        grid=grid,
        in_specs=[_row_spec(32), _row_spec(32), _row_spec(32), _row_spec(8),
                  _full_spec((32, 64)), _full_spec((1, 64)),
                  _full_spec((1, 64)), _full_spec((1, 64))],
        out_specs=[_row_spec(32), _row_spec(32)],
        out_shape=[jax.ShapeDtypeStruct((NP, 32), f32),
                   jax.ShapeDtypeStruct((NP, 32), f32)],
        name="gcn_tc_layer1",
    )(p1a, p1b, xs, dinv8, W1, b1.reshape(1, -1), g1.reshape(1, -1),
      be1.reshape(1, -1))

    p2lo, p2hi = _sc_pass(ys_lo, ys_hi, src2d, dst2d, zeros32,
                          off0=0, cnt0=ROWS, off1=0, cnt1=ROWS)

    out = pl.pallas_call(
        _t3_body,
        grid=grid,
        in_specs=[_row_spec(32), _row_spec(32), _row_spec(32), _row_spec(32),
                  _row_spec(8),
                  _full_spec((64, 128)), _full_spec((1, 128)),
                  _full_spec((1, 128)), _full_spec((1, 128)),
                  _full_spec((128, 128)), _full_spec((1, 128))],
        out_specs=_row_spec(128),
        out_shape=jax.ShapeDtypeStruct((NP, 128), f32),
        name="gcn_tc_layer2_final",
    )(p2lo, p2hi, ys_lo, ys_hi, dinv8, W2, b2.reshape(1, -1),
      g2.reshape(1, -1), be2.reshape(1, -1), Wf, bf.reshape(1, -1))

    return out[:N_NODES]


# speedup vs baseline: 48.1006x; 1.6558x over previous
import functools

import jax
import jax.numpy as jnp
from jax import lax
from jax.experimental import pallas as pl
from jax.experimental.pallas import tpu as pltpu
from jax.experimental.pallas import tpu_sc as plsc

N_NODES = 50000
N_EDGES = 800000
ALPHA = 0.01
EPS = 1e-5

NP = 50176
NQ = NP // 4
ZR = NP // 16
EPAD = 819200
ROWS = EPAD // 128
ROWS_H = ROWS // 2
GB = 8
RBLK = 1568
BN_S = float(1.0 / (1.0 + EPS) ** 0.5)


def _sc_scatter_body(t0, t1, src_hbm, dst_hbm, zeros_hbm, out0, out1,
                     accum, srcb, dstb, rbuf, isem, gsem, ssem,
                     *, off0, cnt0, off1, cnt1):
    c = lax.axis_index("c")
    s = lax.axis_index("s")

    pltpu.sync_copy(zeros_hbm.at[pl.ds(s * ZR, ZR)], accum.at[pl.ds(s * ZR, ZR)])
    plsc.subcore_barrier()

    def do_pass(table, off, cnt):
        per_tile = cnt // 16
        base = off + s * per_tile
        nsc = per_tile // GB

        def load_idx(g, sl):
            pltpu.async_copy(src_hbm.at[pl.ds(base + g * GB, GB)],
                             srcb.at[sl], isem)
            pltpu.async_copy(dst_hbm.at[pl.ds(base + g * GB, GB)],
                             dstb.at[sl], isem)

        def wait_idx(sl):
            for ib in (srcb, dstb):
                pltpu.make_async_copy(src_hbm.at[pl.ds(base, GB)],
                                      ib.at[sl], isem).wait()

        def drain(sem, slot):
            pltpu.make_async_copy(zeros_hbm.at[pl.ds(0, 128)],
                                  rbuf.at[slot], sem).wait()

        def gather(sl, b, slot):
            pltpu.async_copy(table.at[srcb.at[sl].at[b]], rbuf.at[slot], gsem)

        def scatter(sl, b, slot):
            pltpu.async_copy(rbuf.at[slot], accum.at[dstb.at[sl].at[b]],
                             ssem, add=True)

        load_idx(0, 0)

        def g_body(g, carry):
            sl = lax.rem(g, 2)

            @pl.when(g > 0)
            def _():
                drain(gsem, 3)
                scatter(1 - sl, GB - 1, 3)

            wait_idx(sl)

            for b in range(GB):
                slot = b % 4
                if b < 4:
                    @pl.when(g > 0)
                    def _():
                        drain(ssem, slot)
                else:
                    drain(ssem, slot)
                if b == 4:
                    @pl.when(g + 1 < nsc)
                    def _():
                        load_idx(g + 1, 1 - sl)
                gather(sl, b, slot)
                if b > 0:
                    drain(gsem, (b - 1) % 4)
                    scatter(sl, b - 1, (b - 1) % 4)
            return carry

        lax.fori_loop(0, nsc, g_body, 0)

        drain(gsem, 3)
        scatter((nsc - 1) % 2, GB - 1, 3)
        for k in range(4):
            drain(ssem, k)

    @pl.when(c == 0)
    def _():
        do_pass(t0, off0, cnt0)

    @pl.when(c == 1)
    def _():
        do_pass(t1, off1, cnt1)

    plsc.subcore_barrier()

    @pl.when(c == 0)
    def _():
        pltpu.sync_copy(accum.at[pl.ds(s * ZR, ZR)], out0.at[pl.ds(s * ZR, ZR)])

    @pl.when(c == 1)
    def _():
        pltpu.sync_copy(accum.at[pl.ds(s * ZR, ZR)], out1.at[pl.ds(s * ZR, ZR)])


def _sc_pass(t0, t1, src2d, dst2d, zeros, *, off0, cnt0, off1, cnt1):
    f32 = jnp.float32
    body = functools.partial(_sc_scatter_body, off0=off0, cnt0=cnt0, off1=off1,
                             cnt1=cnt1)
    return pl.kernel(
        body,
        out_type=(jax.ShapeDtypeStruct((NP, 32), f32),
                  jax.ShapeDtypeStruct((NP, 32), f32)),
        mesh=plsc.VectorSubcoreMesh(core_axis_name="c", subcore_axis_name="s"),
        scratch_types=(
            pltpu.VMEM_SHARED((NP, 32), f32),
            pltpu.VMEM((2, GB, 128), jnp.int32),
            pltpu.VMEM((2, GB, 128), jnp.int32),
            pltpu.VMEM((4, 128, 32), f32),
            pltpu.SemaphoreType.DMA,
            pltpu.SemaphoreType.DMA,
            pltpu.SemaphoreType.DMA,
        ),
        compiler_params=pltpu.CompilerParams(use_tc_tiling_on_sc=False),
        name=f"gcn_sc_scatter_{cnt0}",
    )(t0, t1, src2d, dst2d, zeros)


def _sc_hist_body(ones_hbm, dst_hbm, zeros_hbm, out0, out1,
                  accum, dstb, srcones, isem, ssem):
    c = lax.axis_index("c")
    s = lax.axis_index("s")

    pltpu.sync_copy(zeros_hbm.at[pl.ds(s * ZR, ZR)], accum.at[pl.ds(s * ZR, ZR)])
    pltpu.sync_copy(ones_hbm, srcones)
    plsc.subcore_barrier()

    per_tile = ROWS_H // 16
    base = c * ROWS_H + s * per_tile
    nsc = per_tile // GB

    def load_idx(g, sl):
        pltpu.async_copy(dst_hbm.at[pl.ds(base + g * GB, GB)],
                         dstb.at[sl], isem)

    def drain_scat():
        pltpu.make_async_copy(zeros_hbm.at[pl.ds(0, 128)], srcones,
                              ssem).wait()

    load_idx(0, 0)

    def g_body(g, carry):
        sl = lax.rem(g, 3)
        pltpu.make_async_copy(dst_hbm.at[pl.ds(base, GB)], dstb.at[sl],
                              isem).wait()
        @pl.when(g + 1 < nsc)
        def _():
            load_idx(g + 1, lax.rem(g + 1, 3))
        for b in range(GB):
            @pl.when(g > 0)
            def _():
                drain_scat()
            pltpu.async_copy(srcones, accum.at[dstb.at[sl].at[b]], ssem,
                             add=True)
        return carry

    lax.fori_loop(0, nsc, g_body, 0)
    for _ in range(GB):
        drain_scat()

    plsc.subcore_barrier()

    @pl.when(c == 0)
    def _():
        pltpu.sync_copy(accum.at[pl.ds(s * ZR, ZR)], out0.at[pl.ds(s * ZR, ZR)])

    @pl.when(c == 1)
    def _():
        pltpu.sync_copy(accum.at[pl.ds(s * ZR, ZR)], out1.at[pl.ds(s * ZR, ZR)])


def _sc_hist(ones32, dst2d, zeros):
    f32 = jnp.float32
    return pl.kernel(
        _sc_hist_body,
        out_type=(jax.ShapeDtypeStruct((NP, 32), f32),
                  jax.ShapeDtypeStruct((NP, 32), f32)),
        mesh=plsc.VectorSubcoreMesh(core_axis_name="c", subcore_axis_name="s"),
        scratch_types=(
            pltpu.VMEM_SHARED((NP, 32), f32),
            pltpu.VMEM((3, GB, 128), jnp.int32),
            pltpu.VMEM((128, 32), f32),
            pltpu.SemaphoreType.DMA,
            pltpu.SemaphoreType.DMA,
        ),
        compiler_params=pltpu.CompilerParams(use_tc_tiling_on_sc=False),
        name="gcn_sc_hist",
    )(ones32, dst2d, zeros)


def _t1_body(emb, dega, degb, xs_out, dinv_out):
    deg = dega[...] + degb[...] + 1.0
    dinv = lax.rsqrt(deg)
    dinv_out[...] = dinv
    xs_out[...] = emb[...] * dinv


def _t2_body(p1a, p1b, xs, dinv, Wbd, g1t, b1t, be1t, lo_out, hi_out):
    d = dinv[...]
    P = (p1a[...] + p1b[...] + xs[...]) * d
    z = jnp.dot(P, Wbd[...], preferred_element_type=jnp.float32)
    A = g1t[...] * BN_S
    y = z * A + (b1t[...] * A + be1t[...])
    y = jnp.where(y >= 0, y, ALPHA * y)
    lo_out[...] = y[:, :128] * d
    hi_out[...] = y[:, 128:] * d


def _t3_body(p2lo, p2hi, yslo, yshi, dinv, W2lo, W2hi, g2t, b2t, be2t,
             Wft, bft, out):
    d = dinv[...]
    Plo = (p2lo[...] + yslo[...]) * d
    Phi = (p2hi[...] + yshi[...]) * d
    z = (jnp.dot(Plo, W2lo[...], preferred_element_type=jnp.float32)
         + jnp.dot(Phi, W2hi[...], preferred_element_type=jnp.float32))
    A = g2t[...] * BN_S
    y = z * A + (b2t[...] * A + be2t[...])
    y = jnp.where(y >= 0, y, ALPHA * y)
    out[...] = (jnp.dot(y, Wft[...], preferred_element_type=jnp.float32)
                + bft[...])


def _row_spec(w):
    return pl.BlockSpec((RBLK, w), lambda i: (i, 0))


def _full_spec(shape):
    nd = len(shape)
    return pl.BlockSpec(shape, lambda i: (0,) * nd)


def _bd4(w):
    return jnp.kron(jnp.eye(4, dtype=w.dtype), w)


def kernel(embeddings, edge_index, W1, b1, g1, be1, W2, b2, g2, be2, Wf, bf):
    f32 = jnp.float32
    grid = (NQ // RBLK,)

    emb128 = jnp.pad(embeddings, ((0, NP - N_NODES), (0, 0))).reshape(NQ, 128)
    src = edge_index[0].astype(jnp.int32)
    dst = edge_index[1].astype(jnp.int32)
    pad_idx = N_NODES + jnp.arange(EPAD - N_EDGES, dtype=jnp.int32) % (NP - N_NODES)
    src2d = jnp.concatenate([src, pad_idx]).reshape(ROWS, 128)
    dst2d = jnp.concatenate([dst, pad_idx]).reshape(ROWS, 128)
    zeros32 = jnp.zeros((NP, 32), f32)
    ones32 = jnp.ones((128, 32), f32)

    dega, degb = _sc_hist(ones32, dst2d, zeros32)

    xs, dinv = pl.pallas_call(
        _t1_body,
        grid=grid,
        in_specs=[_row_spec(128), _row_spec(128), _row_spec(128)],
        out_specs=[_row_spec(128), _row_spec(128)],
        out_shape=[jax.ShapeDtypeStruct((NQ, 128), f32),
                   jax.ShapeDtypeStruct((NQ, 128), f32)],
        name="gcn_tc_prep",
    )(emb128, dega.reshape(NQ, 128), degb.reshape(NQ, 128))

    xs32 = xs.reshape(NP, 32)
    p1a, p1b = _sc_pass(xs32, xs32, src2d, dst2d, zeros32,
                        off0=0, cnt0=ROWS_H, off1=ROWS_H, cnt1=ROWS_H)

    Wbd1 = jnp.concatenate([_bd4(W1[:, :32]), _bd4(W1[:, 32:])], axis=1)
    t4 = lambda v: jnp.tile(v, 4).reshape(1, -1)
    t4s = lambda v: jnp.concatenate([jnp.tile(v[:32], 4),
                                     jnp.tile(v[32:], 4)]).reshape(1, -1)
    ys_lo, ys_hi = pl.pallas_call(
        _t2_body,
        grid=grid,
        in_specs=[_row_spec(128)] * 4 +
                 [_full_spec((128, 256)), _full_spec((1, 256)),
                  _full_spec((1, 256)), _full_spec((1, 256))],
        out_specs=[_row_spec(128), _row_spec(128)],
        out_shape=[jax.ShapeDtypeStruct((NQ, 128), f32),
                   jax.ShapeDtypeStruct((NQ, 128), f32)],
        name="gcn_tc_layer1",
    )(p1a.reshape(NQ, 128), p1b.reshape(NQ, 128), xs, dinv,
      Wbd1, t4s(g1), t4s(b1), t4s(be1))

    p2lo, p2hi = _sc_pass(ys_lo.reshape(NP, 32), ys_hi.reshape(NP, 32),
                          src2d, dst2d, zeros32,
                          off0=0, cnt0=ROWS, off1=0, cnt1=ROWS)

    out_w = pl.pallas_call(
        _t3_body,
        grid=grid,
        in_specs=[_row_spec(128)] * 5 +
                 [_full_spec((128, 512)), _full_spec((128, 512)),
                  _full_spec((1, 512)), _full_spec((1, 512)),
                  _full_spec((1, 512)),
                  _full_spec((512, 512)), _full_spec((1, 512))],
        out_specs=_row_spec(512),
        out_shape=jax.ShapeDtypeStruct((NQ, 512), f32),
        name="gcn_tc_layer2_final",
    )(p2lo.reshape(NQ, 128), p2hi.reshape(NQ, 128),
      ys_lo, ys_hi, dinv,
      _bd4(W2[:32, :]), _bd4(W2[32:, :]),
      t4(g2), t4(b2), t4(be2),
      _bd4(Wf), t4(bf))

    return out_w[:N_NODES // 4].reshape(N_NODES, 128)


# speedup vs baseline: 50.6036x; 1.0520x over previous
import functools

import jax
import jax.numpy as jnp
from jax import lax
from jax.experimental import pallas as pl
from jax.experimental.pallas import tpu as pltpu
from jax.experimental.pallas import tpu_sc as plsc

N_NODES = 50000
N_EDGES = 800000
ALPHA = 0.01
EPS = 1e-5

NP = 50176
NQ = NP // 4
ZR = NP // 16
EPAD = 819200
ROWS = EPAD // 128
ROWS_H = ROWS // 2
GB = 8
RBLK = 1568
BN_S = float(1.0 / (1.0 + EPS) ** 0.5)


def _sc_scatter_body(t0, t1, src_hbm, dst_hbm, zeros_hbm, out0, out1,
                     accum, srcb, dstb, rbuf, isem, gsem, ssem,
                     *, off0, cnt0, off1, cnt1):
    c = lax.axis_index("c")
    s = lax.axis_index("s")

    pltpu.sync_copy(zeros_hbm.at[pl.ds(s * ZR, ZR)], accum.at[pl.ds(s * ZR, ZR)])
    plsc.subcore_barrier()

    def do_pass(table, off, cnt):
        per_tile = cnt // 16
        base = off + s * per_tile
        nsc = per_tile // GB

        def load_idx(g, sl):
            pltpu.async_copy(src_hbm.at[pl.ds(base + g * GB, GB)],
                             srcb.at[sl], isem)
            pltpu.async_copy(dst_hbm.at[pl.ds(base + g * GB, GB)],
                             dstb.at[sl], isem)

        def wait_idx(sl):
            for ib in (srcb, dstb):
                pltpu.make_async_copy(src_hbm.at[pl.ds(base, GB)],
                                      ib.at[sl], isem).wait()

        def drain(sem, slot):
            pltpu.make_async_copy(zeros_hbm.at[pl.ds(0, 128)],
                                  rbuf.at[slot], sem).wait()

        def gather(sl, b, slot):
            pltpu.async_copy(table.at[srcb.at[sl].at[b]], rbuf.at[slot], gsem)

        def scatter(sl, b, slot):
            pltpu.async_copy(rbuf.at[slot], accum.at[dstb.at[sl].at[b]],
                             ssem, add=True)

        load_idx(0, 0)

        def g_body(g, carry):
            sl = lax.rem(g, 2)

            @pl.when(g > 0)
            def _():
                drain(gsem, 3)
                scatter(1 - sl, GB - 1, 3)

            wait_idx(sl)

            for b in range(GB):
                slot = b % 4
                if b < 4:
                    @pl.when(g > 0)
                    def _():
                        drain(ssem, slot)
                else:
                    drain(ssem, slot)
                if b == 4:
                    @pl.when(g + 1 < nsc)
                    def _():
                        load_idx(g + 1, 1 - sl)
                gather(sl, b, slot)
                if b > 0:
                    drain(gsem, (b - 1) % 4)
                    scatter(sl, b - 1, (b - 1) % 4)
            return carry

        lax.fori_loop(0, nsc, g_body, 0)

        drain(gsem, 3)
        scatter((nsc - 1) % 2, GB - 1, 3)
        for k in range(4):
            drain(ssem, k)

    @pl.when(c == 0)
    def _():
        do_pass(t0, off0, cnt0)

    @pl.when(c == 1)
    def _():
        do_pass(t1, off1, cnt1)

    plsc.subcore_barrier()

    @pl.when(c == 0)
    def _():
        pltpu.sync_copy(accum.at[pl.ds(s * ZR, ZR)], out0.at[pl.ds(s * ZR, ZR)])

    @pl.when(c == 1)
    def _():
        pltpu.sync_copy(accum.at[pl.ds(s * ZR, ZR)], out1.at[pl.ds(s * ZR, ZR)])


def _sc_pass(t0, t1, src2d, dst2d, zeros, *, off0, cnt0, off1, cnt1):
    f32 = jnp.float32
    body = functools.partial(_sc_scatter_body, off0=off0, cnt0=cnt0, off1=off1,
                             cnt1=cnt1)
    return pl.kernel(
        body,
        out_type=(jax.ShapeDtypeStruct((NP, 32), f32),
                  jax.ShapeDtypeStruct((NP, 32), f32)),
        mesh=plsc.VectorSubcoreMesh(core_axis_name="c", subcore_axis_name="s"),
        scratch_types=(
            pltpu.VMEM_SHARED((NP, 32), f32),
            pltpu.VMEM((2, GB, 128), jnp.int32),
            pltpu.VMEM((2, GB, 128), jnp.int32),
            pltpu.VMEM((4, 128, 32), f32),
            pltpu.SemaphoreType.DMA,
            pltpu.SemaphoreType.DMA,
            pltpu.SemaphoreType.DMA,
        ),
        compiler_params=pltpu.CompilerParams(use_tc_tiling_on_sc=False),
        name=f"gcn_sc_scatter_{cnt0}",
    )(t0, t1, src2d, dst2d, zeros)


def _sc_hist_body(ones_hbm, dst_hbm, zeros_hbm, out0, out1,
                  accum, dstb, srcones, isem, ssem):
    c = lax.axis_index("c")
    s = lax.axis_index("s")

    pltpu.sync_copy(zeros_hbm.at[pl.ds(s * ZR, ZR)], accum.at[pl.ds(s * ZR, ZR)])
    pltpu.sync_copy(ones_hbm, srcones)
    plsc.subcore_barrier()

    per_tile = ROWS_H // 16
    base = c * ROWS_H + s * per_tile
    nsc = per_tile // GB

    def load_idx(g, sl):
        pltpu.async_copy(dst_hbm.at[pl.ds(base + g * GB, GB)],
                         dstb.at[sl], isem)

    def drain_scat():
        pltpu.make_async_copy(zeros_hbm.at[pl.ds(0, 128)], srcones,
                              ssem).wait()

    load_idx(0, 0)

    def g_body(g, carry):
        sl = lax.rem(g, 3)
        pltpu.make_async_copy(dst_hbm.at[pl.ds(base, GB)], dstb.at[sl],
                              isem).wait()
        @pl.when(g + 1 < nsc)
        def _():
            load_idx(g + 1, lax.rem(g + 1, 3))
        for b in range(GB):
            @pl.when(g > 0)
            def _():
                drain_scat()
            pltpu.async_copy(srcones, accum.at[dstb.at[sl].at[b]], ssem,
                             add=True)
        return carry

    lax.fori_loop(0, nsc, g_body, 0)
    for _ in range(GB):
        drain_scat()

    plsc.subcore_barrier()

    @pl.when(c == 0)
    def _():
        pltpu.sync_copy(accum.at[pl.ds(s * ZR, ZR)], out0.at[pl.ds(s * ZR, ZR)])

    @pl.when(c == 1)
    def _():
        pltpu.sync_copy(accum.at[pl.ds(s * ZR, ZR)], out1.at[pl.ds(s * ZR, ZR)])


def _sc_hist(ones32, dst2d, zeros):
    f32 = jnp.float32
    return pl.kernel(
        _sc_hist_body,
        out_type=(jax.ShapeDtypeStruct((NP, 32), f32),
                  jax.ShapeDtypeStruct((NP, 32), f32)),
        mesh=plsc.VectorSubcoreMesh(core_axis_name="c", subcore_axis_name="s"),
        scratch_types=(
            pltpu.VMEM_SHARED((NP, 32), f32),
            pltpu.VMEM((3, GB, 128), jnp.int32),
            pltpu.VMEM((128, 32), f32),
            pltpu.SemaphoreType.DMA,
            pltpu.SemaphoreType.DMA,
        ),
        compiler_params=pltpu.CompilerParams(use_tc_tiling_on_sc=False),
        name="gcn_sc_hist",
    )(ones32, dst2d, zeros)


def _t1_body(emb, dega, degb, xs_out, dinv_out):
    deg = dega[...] + degb[...] + 1.0
    dinv = lax.rsqrt(deg)
    dinv_out[...] = dinv
    xs_out[...] = emb[...] * dinv


def _t2_body(p1a, p1b, xs, dinv, Wbd, g1t, b1t, be1t, lo_out, hi_out):
    d = dinv[...]
    P = (p1a[...] + p1b[...] + xs[...]) * d
    z = jnp.dot(P, Wbd[...], preferred_element_type=jnp.float32)
    A = g1t[...] * BN_S
    y = z * A + (b1t[...] * A + be1t[...])
    y = jnp.where(y >= 0, y, ALPHA * y)
    lo_out[...] = y[:, :128] * d
    hi_out[...] = y[:, 128:] * d


def _t3_body(p2lo, p2hi, yslo, yshi, dinv, W2lo, W2hi, g2t, b2t, be2t,
             Wft, bft, out):
    d = dinv[...]
    Plo = (p2lo[...] + yslo[...]) * d
    Phi = (p2hi[...] + yshi[...]) * d
    z = (jnp.dot(Plo, W2lo[...], preferred_element_type=jnp.float32)
         + jnp.dot(Phi, W2hi[...], preferred_element_type=jnp.float32))
    A = g2t[...] * BN_S
    y = z * A + (b2t[...] * A + be2t[...])
    y = jnp.where(y >= 0, y, ALPHA * y)
    o = jnp.dot(y, Wft[...], preferred_element_type=jnp.float32) + bft[...]
    out[...] = o.reshape(4 * RBLK, 128)


def _row_spec(w):
    return pl.BlockSpec((RBLK, w), lambda i: (i, 0))


def _full_spec(shape):
    nd = len(shape)
    return pl.BlockSpec(shape, lambda i: (0,) * nd)


def _bd4(w):
    return jnp.kron(jnp.eye(4, dtype=w.dtype), w)


def kernel(embeddings, edge_index, W1, b1, g1, be1, W2, b2, g2, be2, Wf, bf):
    f32 = jnp.float32
    grid = (NQ // RBLK,)

    emb128 = jnp.pad(embeddings, ((0, NP - N_NODES), (0, 0))).reshape(NQ, 128)
    src = edge_index[0].astype(jnp.int32)
    dst = edge_index[1].astype(jnp.int32)
    pad_idx = N_NODES + jnp.arange(EPAD - N_EDGES, dtype=jnp.int32) % (NP - N_NODES)
    src2d = jnp.concatenate([src, pad_idx]).reshape(ROWS, 128)
    dst2d = jnp.concatenate([dst, pad_idx]).reshape(ROWS, 128)
    zeros32 = jnp.zeros((NP, 32), f32)
    ones32 = jnp.ones((128, 32), f32)

    dega, degb = _sc_hist(ones32, dst2d, zeros32)

    xs, dinv = pl.pallas_call(
        _t1_body,
        grid=grid,
        in_specs=[_row_spec(128), _row_spec(128), _row_spec(128)],
        out_specs=[_row_spec(128), _row_spec(128)],
        out_shape=[jax.ShapeDtypeStruct((NQ, 128), f32),
                   jax.ShapeDtypeStruct((NQ, 128), f32)],
        name="gcn_tc_prep",
    )(emb128, dega.reshape(NQ, 128), degb.reshape(NQ, 128))

    xs32 = xs.reshape(NP, 32)
    p1a, p1b = _sc_pass(xs32, xs32, src2d, dst2d, zeros32,
                        off0=0, cnt0=ROWS_H, off1=ROWS_H, cnt1=ROWS_H)

    Wbd1 = jnp.concatenate([_bd4(W1[:, :32]), _bd4(W1[:, 32:])], axis=1)
    t4 = lambda v: jnp.tile(v, 4).reshape(1, -1)
    t4s = lambda v: jnp.concatenate([jnp.tile(v[:32], 4),
                                     jnp.tile(v[32:], 4)]).reshape(1, -1)
    ys_lo, ys_hi = pl.pallas_call(
        _t2_body,
        grid=grid,
        in_specs=[_row_spec(128)] * 4 +
                 [_full_spec((128, 256)), _full_spec((1, 256)),
                  _full_spec((1, 256)), _full_spec((1, 256))],
        out_specs=[_row_spec(128), _row_spec(128)],
        out_shape=[jax.ShapeDtypeStruct((NQ, 128), f32),
                   jax.ShapeDtypeStruct((NQ, 128), f32)],
        name="gcn_tc_layer1",
    )(p1a.reshape(NQ, 128), p1b.reshape(NQ, 128), xs, dinv,
      Wbd1, t4s(g1), t4s(b1), t4s(be1))

    p2lo, p2hi = _sc_pass(ys_lo.reshape(NP, 32), ys_hi.reshape(NP, 32),
                          src2d, dst2d, zeros32,
                          off0=0, cnt0=ROWS, off1=0, cnt1=ROWS)

    out_w = pl.pallas_call(
        _t3_body,
        grid=grid,
        in_specs=[_row_spec(128)] * 5 +
                 [_full_spec((128, 512)), _full_spec((128, 512)),
                  _full_spec((1, 512)), _full_spec((1, 512)),
                  _full_spec((1, 512)),
                  _full_spec((512, 512)), _full_spec((1, 512))],
        out_specs=pl.BlockSpec((4 * RBLK, 128), lambda i: (i, 0)),
        out_shape=jax.ShapeDtypeStruct((NP, 128), f32),
        name="gcn_tc_layer2_final",
    )(p2lo.reshape(NQ, 128), p2hi.reshape(NQ, 128),
      ys_lo, ys_hi, dinv,
      _bd4(W2[:32, :]), _bd4(W2[32:, :]),
      t4(g2), t4(b2), t4(be2),
      _bd4(Wf), t4(bf))

    return out_w[:N_NODES]


# speedup vs baseline: 52.4913x; 1.0373x over previous
import functools

import jax
import jax.numpy as jnp
from jax import lax
from jax.experimental import pallas as pl
from jax.experimental.pallas import tpu as pltpu
from jax.experimental.pallas import tpu_sc as plsc

N_NODES = 50000
N_EDGES = 800000
ALPHA = 0.01
EPS = 1e-5

NP = 50176
NQ = NP // 4
ZR = NP // 16
EPAD = 819200
ROWS = EPAD // 128
ROWS_H = ROWS // 2
GB = 8
RBLK = 1568
BN_S = float(1.0 / (1.0 + EPS) ** 0.5)


def _sc_scatter_body(t0, t1, src_hbm, dst_hbm, zeros_hbm, out0, out1,
                     accum, srcb, dstb, rbuf, isem, gsem, ssem,
                     *, off0, cnt0, off1, cnt1):
    c = lax.axis_index("c")
    s = lax.axis_index("s")

    pltpu.sync_copy(zeros_hbm.at[pl.ds(s * ZR, ZR)], accum.at[pl.ds(s * ZR, ZR)])
    plsc.subcore_barrier()

    def do_pass(table, off, cnt):
        per_tile = cnt // 16
        base = off + s * per_tile
        nsc = per_tile // GB

        def load_idx(g, sl):
            pltpu.async_copy(src_hbm.at[pl.ds(base + g * GB, GB)],
                             srcb.at[sl], isem)
            pltpu.async_copy(dst_hbm.at[pl.ds(base + g * GB, GB)],
                             dstb.at[sl], isem)

        def wait_idx(sl):
            for ib in (srcb, dstb):
                pltpu.make_async_copy(src_hbm.at[pl.ds(base, GB)],
                                      ib.at[sl], isem).wait()

        def drain(sem, slot):
            pltpu.make_async_copy(zeros_hbm.at[pl.ds(0, 128)],
                                  rbuf.at[slot], sem).wait()

        def gather(sl, b, slot):
            pltpu.async_copy(table.at[srcb.at[sl].at[b]], rbuf.at[slot], gsem)

        def scatter(sl, b, slot):
            pltpu.async_copy(rbuf.at[slot], accum.at[dstb.at[sl].at[b]],
                             ssem, add=True)

        load_idx(0, 0)

        def g_body(g, carry):
            sl = lax.rem(g, 2)

            @pl.when(g > 0)
            def _():
                drain(gsem, 3)
                scatter(1 - sl, GB - 1, 3)

            wait_idx(sl)

            for b in range(GB):
                slot = b % 4
                if b < 4:
                    @pl.when(g > 0)
                    def _():
                        drain(ssem, slot)
                else:
                    drain(ssem, slot)
                if b == 4:
                    @pl.when(g + 1 < nsc)
                    def _():
                        load_idx(g + 1, 1 - sl)
                gather(sl, b, slot)
                if b > 0:
                    drain(gsem, (b - 1) % 4)
                    scatter(sl, b - 1, (b - 1) % 4)
            return carry

        lax.fori_loop(0, nsc, g_body, 0)

        drain(gsem, 3)
        scatter((nsc - 1) % 2, GB - 1, 3)
        for k in range(4):
            drain(ssem, k)

    @pl.when(c == 0)
    def _():
        do_pass(t0, off0, cnt0)

    @pl.when(c == 1)
    def _():
        do_pass(t1, off1, cnt1)

    plsc.subcore_barrier()

    @pl.when(c == 0)
    def _():
        pltpu.sync_copy(accum.at[pl.ds(s * ZR, ZR)], out0.at[pl.ds(s * ZR, ZR)])

    @pl.when(c == 1)
    def _():
        pltpu.sync_copy(accum.at[pl.ds(s * ZR, ZR)], out1.at[pl.ds(s * ZR, ZR)])


def _sc_pass(t0, t1, src2d, dst2d, zeros, *, off0, cnt0, off1, cnt1):
    f32 = jnp.float32
    body = functools.partial(_sc_scatter_body, off0=off0, cnt0=cnt0, off1=off1,
                             cnt1=cnt1)
    return pl.kernel(
        body,
        out_type=(jax.ShapeDtypeStruct((NP, 32), f32),
                  jax.ShapeDtypeStruct((NP, 32), f32)),
        mesh=plsc.VectorSubcoreMesh(core_axis_name="c", subcore_axis_name="s"),
        scratch_types=(
            pltpu.VMEM_SHARED((NP, 32), f32),
            pltpu.VMEM((2, GB, 128), jnp.int32),
            pltpu.VMEM((2, GB, 128), jnp.int32),
            pltpu.VMEM((4, 128, 32), f32),
            pltpu.SemaphoreType.DMA,
            pltpu.SemaphoreType.DMA,
            pltpu.SemaphoreType.DMA,
        ),
        compiler_params=pltpu.CompilerParams(use_tc_tiling_on_sc=False),
        name=f"gcn_sc_scatter_{cnt0}",
    )(t0, t1, src2d, dst2d, zeros)


def _sc_hist_body(ones_hbm, dst_hbm, zeros_hbm, out0, out1,
                  accum, dstb, srcones, isem, ssem):
    c = lax.axis_index("c")
    s = lax.axis_index("s")

    pltpu.sync_copy(zeros_hbm.at[pl.ds(s * ZR, ZR)], accum.at[pl.ds(s * ZR, ZR)])
    pltpu.sync_copy(ones_hbm, srcones)
    plsc.subcore_barrier()

    per_tile = ROWS_H // 16
    base = c * ROWS_H + s * per_tile
    nsc = per_tile // GB

    def load_idx(g, sl):
        pltpu.async_copy(dst_hbm.at[pl.ds(base + g * GB, GB)],
                         dstb.at[sl], isem)

    def drain_scat():
        pltpu.make_async_copy(zeros_hbm.at[pl.ds(0, 128)], srcones,
                              ssem).wait()

    load_idx(0, 0)

    def g_body(g, carry):
        sl = lax.rem(g, 3)
        pltpu.make_async_copy(dst_hbm.at[pl.ds(base, GB)], dstb.at[sl],
                              isem).wait()
        @pl.when(g + 1 < nsc)
        def _():
            load_idx(g + 1, lax.rem(g + 1, 3))
        for b in range(GB):
            @pl.when(g > 0)
            def _():
                drain_scat()
            pltpu.async_copy(srcones, accum.at[dstb.at[sl].at[b]], ssem,
                             add=True)
        return carry

    lax.fori_loop(0, nsc, g_body, 0)
    for _ in range(GB):
        drain_scat()

    plsc.subcore_barrier()

    @pl.when(c == 0)
    def _():
        pltpu.sync_copy(accum.at[pl.ds(s * ZR, ZR)], out0.at[pl.ds(s * ZR, ZR)])

    @pl.when(c == 1)
    def _():
        pltpu.sync_copy(accum.at[pl.ds(s * ZR, ZR)], out1.at[pl.ds(s * ZR, ZR)])


def _sc_hist(ones32, dst2d, zeros):
    f32 = jnp.float32
    return pl.kernel(
        _sc_hist_body,
        out_type=(jax.ShapeDtypeStruct((NP, 32), f32),
                  jax.ShapeDtypeStruct((NP, 32), f32)),
        mesh=plsc.VectorSubcoreMesh(core_axis_name="c", subcore_axis_name="s"),
        scratch_types=(
            pltpu.VMEM_SHARED((NP, 32), f32),
            pltpu.VMEM((3, GB, 128), jnp.int32),
            pltpu.VMEM((128, 32), f32),
            pltpu.SemaphoreType.DMA,
            pltpu.SemaphoreType.DMA,
        ),
        compiler_params=pltpu.CompilerParams(use_tc_tiling_on_sc=False),
        name="gcn_sc_hist",
    )(ones32, dst2d, zeros)


def _t1_body(emb, dega, degb, xs_out, dinv_out):
    deg = dega[...] + degb[...] + 1.0
    dinv = lax.rsqrt(deg)
    dinv_out[...] = dinv
    xs_out[...] = emb[...] * dinv


def _t2_body(p1a, p1b, xs, dinv, Wbd, g1t, b1t, be1t, lo_out, hi_out):
    d = dinv[...]
    P = (p1a[...] + p1b[...] + xs[...]) * d
    z = jnp.dot(P, Wbd[...], preferred_element_type=jnp.float32)
    A = g1t[...] * BN_S
    y = z * A + (b1t[...] * A + be1t[...])
    y = jnp.where(y >= 0, y, ALPHA * y)
    lo_out[...] = y[:, :128] * d
    hi_out[...] = y[:, 128:] * d


def _t3_body(p2lo, p2hi, yslo, yshi, dinv, W2lo, W2hi, g2t, b2t, be2t,
             Wft, bft, out):
    d = dinv[...]
    Plo = (p2lo[...] + yslo[...]) * d
    Phi = (p2hi[...] + yshi[...]) * d
    z = (jnp.dot(Plo, W2lo[...], preferred_element_type=jnp.float32)
         + jnp.dot(Phi, W2hi[...], preferred_element_type=jnp.float32))
    A = g2t[...] * BN_S
    y = z * A + (b2t[...] * A + be2t[...])
    y = jnp.where(y >= 0, y, ALPHA * y)
    o = jnp.dot(y, Wft[...], preferred_element_type=jnp.float32) + bft[...]
    out[...] = o.reshape(4 * RBLK, 128)


def _row_spec(w):
    return pl.BlockSpec((RBLK, w), lambda i: (i, 0))


def _full_spec(shape):
    nd = len(shape)
    return pl.BlockSpec(shape, lambda i: (0,) * nd)


def _bd4(w):
    return jnp.kron(jnp.eye(4, dtype=w.dtype), w)


def kernel(embeddings, edge_index, W1, b1, g1, be1, W2, b2, g2, be2, Wf, bf):
    f32 = jnp.float32
    grid = (NQ // RBLK,)

    emb128 = jnp.pad(embeddings, ((0, NP - N_NODES), (0, 0))).reshape(NQ, 128)
    src = edge_index[0].astype(jnp.int32)
    dst = edge_index[1].astype(jnp.int32)
    pad_idx = N_NODES + jnp.arange(EPAD - N_EDGES, dtype=jnp.int32) % (NP - N_NODES)
    src2d = jnp.concatenate([src, pad_idx]).reshape(ROWS, 128)
    dst2d = jnp.concatenate([dst, pad_idx]).reshape(ROWS, 128)
    zeros32 = jnp.zeros((NP, 32), f32)
    ones32 = jnp.ones((128, 32), f32)

    dega, degb = _sc_hist(ones32, dst2d, zeros32)

    xs, dinv = pl.pallas_call(
        _t1_body,
        grid=grid,
        in_specs=[_row_spec(128), _row_spec(128), _row_spec(128)],
        out_specs=[_row_spec(128), _row_spec(128)],
        out_shape=[jax.ShapeDtypeStruct((NQ, 128), f32),
                   jax.ShapeDtypeStruct((NQ, 128), f32)],
        name="gcn_tc_prep",
    )(emb128, dega.reshape(NQ, 128), degb.reshape(NQ, 128))

    xs32 = xs.reshape(NP, 32)
    p1a, p1b = _sc_pass(xs32, xs32, src2d, dst2d, zeros32,
                        off0=0, cnt0=ROWS_H, off1=ROWS_H, cnt1=ROWS_H)

    Wbd1 = jnp.concatenate([_bd4(W1[:, :32]), _bd4(W1[:, 32:])], axis=1)
    t4 = lambda v: jnp.tile(v, 4).reshape(1, -1)
    t4s = lambda v: jnp.concatenate([jnp.tile(v[:32], 4),
                                     jnp.tile(v[32:], 4)]).reshape(1, -1)
    ys_lo, ys_hi = pl.pallas_call(
        _t2_body,
        grid=grid,
        in_specs=[_row_spec(128)] * 4 +
                 [_full_spec((128, 256)), _full_spec((1, 256)),
                  _full_spec((1, 256)), _full_spec((1, 256))],
        out_specs=[_row_spec(128), _row_spec(128)],
        out_shape=[jax.ShapeDtypeStruct((NQ, 128), f32),
                   jax.ShapeDtypeStruct((NQ, 128), f32)],
        name="gcn_tc_layer1",
    )(p1a.reshape(NQ, 128), p1b.reshape(NQ, 128), xs, dinv,
      Wbd1, t4s(g1), t4s(b1), t4s(be1))

    p2lo, p2hi = _sc_pass(ys_lo.reshape(NP, 32), ys_hi.reshape(NP, 32),
                          src2d, dst2d, zeros32,
                          off0=0, cnt0=ROWS, off1=0, cnt1=ROWS)

    out_w = pl.pallas_call(
        _t3_body,
        grid=grid,
        in_specs=[_row_spec(128)] * 5 +
                 [_full_spec((128, 512)), _full_spec((128, 512)),
                  _full_spec((1, 512)), _full_spec((1, 512)),
                  _full_spec((1, 512)),
                  _full_spec((512, 512)), _full_spec((1, 512))],
        out_specs=pl.BlockSpec((4 * RBLK, 128), lambda i: (i, 0)),
        out_shape=jax.ShapeDtypeStruct((N_NODES, 128), f32),
        name="gcn_tc_layer2_final",
    )(p2lo.reshape(NQ, 128), p2hi.reshape(NQ, 128),
      ys_lo, ys_hi, dinv,
      _bd4(W2[:32, :]), _bd4(W2[32:, :]),
      t4(g2), t4(b2), t4(be2),
      _bd4(Wf), t4(bf))

    return out_w


# speedup vs baseline: 56.2373x; 1.0714x over previous
import functools

import jax
import jax.numpy as jnp
from jax import lax
from jax.experimental import pallas as pl
from jax.experimental.pallas import tpu as pltpu
from jax.experimental.pallas import tpu_sc as plsc

N_NODES = 50000
N_EDGES = 800000
ALPHA = 0.01
EPS = 1e-5

NP = 50176
NQ = NP // 4
ZR = NP // 16
EROWS = N_EDGES // 128
GB = 8
NSC_FULL = EROWS // GB
RBLK = 1568
BN_S = float(1.0 / (1.0 + EPS) ** 0.5)


def _tile_range(pass_base, count, s):
    per = count // 16
    rem = count - 16 * per
    nsc = per + jnp.where(s < rem, 1, 0)
    base_sc = pass_base + per * s + jnp.minimum(s, rem)
    return base_sc, nsc


def _sc_scatter_body(t0, t1, edge_hbm, tsrc_hbm, tdst_hbm, zeros_hbm,
                     out0, out1, accum, srcb, dstb, rbuf, isem, gsem, ssem,
                     *, base0, cnt0, tail0, base1, cnt1, tail1):
    c = lax.axis_index("c")
    s = lax.axis_index("s")
    src_hbm = edge_hbm.at[0]
    dst_hbm = edge_hbm.at[1]

    pltpu.sync_copy(zeros_hbm.at[pl.ds(s * ZR, ZR)], accum.at[pl.ds(s * ZR, ZR)])
    plsc.subcore_barrier()

    def drain(sem, slot):
        pltpu.make_async_copy(zeros_hbm.at[pl.ds(0, 128)],
                              rbuf.at[slot], sem).wait()

    def do_pass(table, pass_base, count):
        base_sc, nsc = _tile_range(pass_base, count, s)
        base = base_sc * GB

        def load_idx(g, sl):
            pltpu.async_copy(src_hbm.at[pl.ds(base + g * GB, GB)],
                             srcb.at[sl], isem)
            pltpu.async_copy(dst_hbm.at[pl.ds(base + g * GB, GB)],
                             dstb.at[sl], isem)

        def wait_idx(sl):
            for ib in (srcb, dstb):
                pltpu.make_async_copy(src_hbm.at[pl.ds(base, GB)],
                                      ib.at[sl], isem).wait()

        def gather(sl, b, slot):
            pltpu.async_copy(table.at[srcb.at[sl].at[b]], rbuf.at[slot], gsem)

        def scatter(sl, b, slot):
            pltpu.async_copy(rbuf.at[slot], accum.at[dstb.at[sl].at[b]],
                             ssem, add=True)

        load_idx(0, 0)

        def g_body(g, carry):
            sl = lax.rem(g, 2)

            @pl.when(g > 0)
            def _():
                drain(gsem, 3)
                scatter(1 - sl, GB - 1, 3)

            wait_idx(sl)

            for b in range(GB):
                slot = b % 4
                if b < 4:
                    @pl.when(g > 0)
                    def _():
                        drain(ssem, slot)
                else:
                    drain(ssem, slot)
                if b == 4:
                    @pl.when(g + 1 < nsc)
                    def _():
                        load_idx(g + 1, 1 - sl)
                gather(sl, b, slot)
                if b > 0:
                    drain(gsem, (b - 1) % 4)
                    scatter(sl, b - 1, (b - 1) % 4)
            return carry

        lax.fori_loop(0, nsc, g_body, 0)

        drain(gsem, 3)
        scatter(lax.rem(nsc - 1, 2), GB - 1, 3)
        for k in range(4):
            drain(ssem, k)

    def do_tail(table):
        pltpu.sync_copy(tsrc_hbm, srcb.at[0])
        pltpu.sync_copy(tdst_hbm, dstb.at[0])
        for b in range(GB):
            pltpu.async_copy(table.at[srcb.at[0].at[b]], rbuf.at[0],
                             gsem).wait()
            pltpu.async_copy(rbuf.at[0], accum.at[dstb.at[0].at[b]],
                             ssem, add=True).wait()

    @pl.when(c == 0)
    def _():
        do_pass(t0, base0, cnt0)
        if tail0:
            @pl.when(s == 15)
            def _():
                do_tail(t0)

    @pl.when(c == 1)
    def _():
        do_pass(t1, base1, cnt1)
        if tail1:
            @pl.when(s == 15)
            def _():
                do_tail(t1)

    plsc.subcore_barrier()

    @pl.when(c == 0)
    def _():
        pltpu.sync_copy(accum.at[pl.ds(s * ZR, ZR)], out0.at[pl.ds(s * ZR, ZR)])

    @pl.when(c == 1)
    def _():
        pltpu.sync_copy(accum.at[pl.ds(s * ZR, ZR)], out1.at[pl.ds(s * ZR, ZR)])


def _sc_pass(t0, t1, edge3d, tsrc, tdst, zeros, **split):
    f32 = jnp.float32
    body = functools.partial(_sc_scatter_body, **split)
    return pl.kernel(
        body,
        out_type=(jax.ShapeDtypeStruct((NP, 32), f32),
                  jax.ShapeDtypeStruct((NP, 32), f32)),
        mesh=plsc.VectorSubcoreMesh(core_axis_name="c", subcore_axis_name="s"),
        scratch_types=(
            pltpu.VMEM_SHARED((NP, 32), f32),
            pltpu.VMEM((2, GB, 128), jnp.int32),
            pltpu.VMEM((2, GB, 128), jnp.int32),
            pltpu.VMEM((4, 128, 32), f32),
            pltpu.SemaphoreType.DMA,
            pltpu.SemaphoreType.DMA,
            pltpu.SemaphoreType.DMA,
        ),
        compiler_params=pltpu.CompilerParams(use_tc_tiling_on_sc=False),
        name=f"gcn_sc_scatter_{split['cnt0']}",
    )(t0, t1, edge3d, tsrc, tdst, zeros)


def _sc_hist_body(ones_hbm, edge_hbm, tdst_hbm, zeros_hbm, out0, out1,
                  accum, dstb, srcones, isem, ssem):
    c = lax.axis_index("c")
    s = lax.axis_index("s")
    dst_hbm = edge_hbm.at[1]

    pltpu.sync_copy(zeros_hbm.at[pl.ds(s * ZR, ZR)], accum.at[pl.ds(s * ZR, ZR)])
    pltpu.sync_copy(ones_hbm, srcones)
    plsc.subcore_barrier()

    half = NSC_FULL // 2
    base_sc, nsc = _tile_range(half * c, half + c, s)
    base = base_sc * GB

    def load_idx(g, sl):
        pltpu.async_copy(dst_hbm.at[pl.ds(base + g * GB, GB)],
                         dstb.at[sl], isem)

    def drain_scat():
        pltpu.make_async_copy(zeros_hbm.at[pl.ds(0, 128)], srcones,
                              ssem).wait()

    load_idx(0, 0)

    def g_body(g, carry):
        sl = lax.rem(g, 3)
        pltpu.make_async_copy(dst_hbm.at[pl.ds(base, GB)], dstb.at[sl],
                              isem).wait()
        @pl.when(g + 1 < nsc)
        def _():
            load_idx(g + 1, lax.rem(g + 1, 3))
        for b in range(GB):
            @pl.when(g > 0)
            def _():
                drain_scat()
            pltpu.async_copy(srcones, accum.at[dstb.at[sl].at[b]], ssem,
                             add=True)
        return carry

    lax.fori_loop(0, nsc, g_body, 0)
    for _ in range(GB):
        drain_scat()

    @pl.when((c == 1) & (s == 15))
    def _():
        pltpu.sync_copy(tdst_hbm, dstb.at[0])
        for b in range(GB):
            pltpu.async_copy(srcones, accum.at[dstb.at[0].at[b]], ssem,
                             add=True).wait()

    plsc.subcore_barrier()

    @pl.when(c == 0)
    def _():
        pltpu.sync_copy(accum.at[pl.ds(s * ZR, ZR)], out0.at[pl.ds(s * ZR, ZR)])

    @pl.when(c == 1)
    def _():
        pltpu.sync_copy(accum.at[pl.ds(s * ZR, ZR)], out1.at[pl.ds(s * ZR, ZR)])


def _sc_hist(ones32, edge3d, tdst, zeros):
    f32 = jnp.float32
    return pl.kernel(
        _sc_hist_body,
        out_type=(jax.ShapeDtypeStruct((NP, 32), f32),
                  jax.ShapeDtypeStruct((NP, 32), f32)),
        mesh=plsc.VectorSubcoreMesh(core_axis_name="c", subcore_axis_name="s"),
        scratch_types=(
            pltpu.VMEM_SHARED((NP, 32), f32),
            pltpu.VMEM((3, GB, 128), jnp.int32),
            pltpu.VMEM((128, 32), f32),
            pltpu.SemaphoreType.DMA,
            pltpu.SemaphoreType.DMA,
        ),
        compiler_params=pltpu.CompilerParams(use_tc_tiling_on_sc=False),
        name="gcn_sc_hist",
    )(ones32, edge3d, tdst, zeros)


def _t1_body(emb, dega, degb, xs_out, dinv_out):
    deg = dega[...] + degb[...] + 1.0
    dinv = lax.rsqrt(deg)
    dinv_out[...] = dinv
    xs_out[...] = emb[...] * dinv


def _t2_body(p1a, p1b, xs, dinv, Wbd, g1t, b1t, be1t, lo_out, hi_out):
    d = dinv[...]
    P = (p1a[...] + p1b[...] + xs[...]) * d
    z = jnp.dot(P, Wbd[...], preferred_element_type=jnp.float32)
    A = g1t[...] * BN_S
    y = z * A + (b1t[...] * A + be1t[...])
    y = jnp.where(y >= 0, y, ALPHA * y)
    lo_out[...] = y[:, :128] * d
    hi_out[...] = y[:, 128:] * d


def _t3_body(p2lo, p2hi, yslo, yshi, dinv, W2lo, W2hi, g2t, b2t, be2t,
             Wft, bft, out):
    d = dinv[...]
    Plo = (p2lo[...] + yslo[...]) * d
    Phi = (p2hi[...] + yshi[...]) * d
    z = (jnp.dot(Plo, W2lo[...], preferred_element_type=jnp.float32)
         + jnp.dot(Phi, W2hi[...], preferred_element_type=jnp.float32))
    A = g2t[...] * BN_S
    y = z * A + (b2t[...] * A + be2t[...])
    y = jnp.where(y >= 0, y, ALPHA * y)
    o = jnp.dot(y, Wft[...], preferred_element_type=jnp.float32) + bft[...]
    out[...] = o.reshape(4 * RBLK, 128)


def _row_spec(w):
    return pl.BlockSpec((RBLK, w), lambda i: (i, 0))


def _full_spec(shape):
    nd = len(shape)
    return pl.BlockSpec(shape, lambda i: (0,) * nd)


def _bd4(w):
    return jnp.kron(jnp.eye(4, dtype=w.dtype), w)


def kernel(embeddings, edge_index, W1, b1, g1, be1, W2, b2, g2, be2, Wf, bf):
    f32 = jnp.float32
    i32 = jnp.int32
    grid = (NQ // RBLK,)

    emb128 = embeddings.reshape(N_NODES * 32 // 128, 128)
    edge3d = edge_index.astype(i32).reshape(2, EROWS, 128)
    trash = N_NODES + jnp.arange(768, dtype=i32) % (NP - N_NODES)
    tail_at = NSC_FULL * GB * 128
    tsrc = jnp.concatenate([edge_index[0, tail_at:].astype(i32),
                            trash]).reshape(GB, 128)
    tdst = jnp.concatenate([edge_index[1, tail_at:].astype(i32),
                            trash]).reshape(GB, 128)
    zeros32 = jnp.zeros((NP, 32), f32)
    ones32 = jnp.ones((128, 32), f32)

    dega, degb = _sc_hist(ones32, edge3d, tdst, zeros32)

    xs, dinv = pl.pallas_call(
        _t1_body,
        grid=grid,
        in_specs=[_row_spec(128), _row_spec(128), _row_spec(128)],
        out_specs=[_row_spec(128), _row_spec(128)],
        out_shape=[jax.ShapeDtypeStruct((NQ, 128), f32),
                   jax.ShapeDtypeStruct((NQ, 128), f32)],
        name="gcn_tc_prep",
    )(emb128, dega.reshape(NQ, 128), degb.reshape(NQ, 128))

    xs32 = xs.reshape(NP, 32)
    half = NSC_FULL // 2
    p1a, p1b = _sc_pass(xs32, xs32, edge3d, tsrc, tdst, zeros32,
                        base0=0, cnt0=half, tail0=False,
                        base1=half, cnt1=half + 1, tail1=True)

    Wbd1 = jnp.concatenate([_bd4(W1[:, :32]), _bd4(W1[:, 32:])], axis=1)
    t4 = lambda v: jnp.tile(v, 4).reshape(1, -1)
    t4s = lambda v: jnp.concatenate([jnp.tile(v[:32], 4),
                                     jnp.tile(v[32:], 4)]).reshape(1, -1)
    ys_lo, ys_hi = pl.pallas_call(
        _t2_body,
        grid=grid,
        in_specs=[_row_spec(128)] * 4 +
                 [_full_spec((128, 256)), _full_spec((1, 256)),
                  _full_spec((1, 256)), _full_spec((1, 256))],
        out_specs=[_row_spec(128), _row_spec(128)],
        out_shape=[jax.ShapeDtypeStruct((NQ, 128), f32),
                   jax.ShapeDtypeStruct((NQ, 128), f32)],
        name="gcn_tc_layer1",
    )(p1a.reshape(NQ, 128), p1b.reshape(NQ, 128), xs, dinv,
      Wbd1, t4s(g1), t4s(b1), t4s(be1))

    p2lo, p2hi = _sc_pass(ys_lo.reshape(NP, 32), ys_hi.reshape(NP, 32),
                          edge3d, tsrc, tdst, zeros32,
                          base0=0, cnt0=NSC_FULL, tail0=True,
                          base1=0, cnt1=NSC_FULL, tail1=True)

    out_w = pl.pallas_call(
        _t3_body,
        grid=grid,
        in_specs=[_row_spec(128)] * 5 +
                 [_full_spec((128, 512)), _full_spec((128, 512)),
                  _full_spec((1, 512)), _full_spec((1, 512)),
                  _full_spec((1, 512)),
                  _full_spec((512, 512)), _full_spec((1, 512))],
        out_specs=pl.BlockSpec((4 * RBLK, 128), lambda i: (i, 0)),
        out_shape=jax.ShapeDtypeStruct((N_NODES, 128), f32),
        name="gcn_tc_layer2_final",
    )(p2lo.reshape(NQ, 128), p2hi.reshape(NQ, 128),
      ys_lo, ys_hi, dinv,
      _bd4(W2[:32, :]), _bd4(W2[32:, :]),
      t4(g2), t4(b2), t4(be2),
      _bd4(Wf), t4(bf))

    return out_w


# speedup vs baseline: 56.2886x; 1.0009x over previous
import functools

import jax
import jax.numpy as jnp
from jax import lax
from jax.experimental import pallas as pl
from jax.experimental.pallas import tpu as pltpu
from jax.experimental.pallas import tpu_sc as plsc

N_NODES = 50000
N_EDGES = 800000
ALPHA = 0.01
EPS = 1e-5

NP = 50176
NQ = NP // 4
ZR = NP // 16
EROWS = N_EDGES // 128
GB = 8
NSC_FULL = EROWS // GB
RBLK = 1568
BN_S = float(1.0 / (1.0 + EPS) ** 0.5)


def _tile_range(pass_base, count, s):
    per = count // 16
    rem = count - 16 * per
    nsc = per + jnp.where(s < rem, 1, 0)
    base_sc = pass_base + per * s + jnp.minimum(s, rem)
    return base_sc, nsc


def _sc_scatter_body(t0, t1, edge_hbm, tsrc_hbm, tdst_hbm, zeros_hbm,
                     out0, out1, accum, srcb, dstb, rbuf, isem, gsem, ssem,
                     *, base0, cnt0, tail0, base1, cnt1, tail1):
    c = lax.axis_index("c")
    s = lax.axis_index("s")
    src_hbm = edge_hbm.at[0]
    dst_hbm = edge_hbm.at[1]

    pltpu.sync_copy(zeros_hbm.at[pl.ds(s * ZR, ZR)], accum.at[pl.ds(s * ZR, ZR)])
    plsc.subcore_barrier()

    def drain(sem, slot):
        pltpu.make_async_copy(zeros_hbm.at[pl.ds(0, 128)],
                              rbuf.at[slot], sem).wait()

    def do_pass(table, pass_base, count):
        base_sc, nsc = _tile_range(pass_base, count, s)
        base = base_sc * GB

        def load_idx(g, sl):
            pltpu.async_copy(src_hbm.at[pl.ds(base + g * GB, GB)],
                             srcb.at[sl], isem)
            pltpu.async_copy(dst_hbm.at[pl.ds(base + g * GB, GB)],
                             dstb.at[sl], isem)

        def wait_idx(sl):
            for ib in (srcb, dstb):
                pltpu.make_async_copy(src_hbm.at[pl.ds(base, GB)],
                                      ib.at[sl], isem).wait()

        def gather(sl, b, slot):
            pltpu.async_copy(table.at[srcb.at[sl].at[b]], rbuf.at[slot], gsem)

        def scatter(sl, b, slot):
            pltpu.async_copy(rbuf.at[slot], accum.at[dstb.at[sl].at[b]],
                             ssem, add=True)

        load_idx(0, 0)

        def g_body(g, carry):
            sl = lax.rem(g, 2)

            @pl.when(g > 0)
            def _():
                drain(gsem, 3)
                scatter(1 - sl, GB - 1, 3)

            wait_idx(sl)

            for b in range(GB):
                slot = b % 4
                if b < 4:
                    @pl.when(g > 0)
                    def _():
                        drain(ssem, slot)
                else:
                    drain(ssem, slot)
                if b == 4:
                    @pl.when(g + 1 < nsc)
                    def _():
                        load_idx(g + 1, 1 - sl)
                gather(sl, b, slot)
                if b > 0:
                    drain(gsem, (b - 1) % 4)
                    scatter(sl, b - 1, (b - 1) % 4)
            return carry

        lax.fori_loop(0, nsc, g_body, 0)

        drain(gsem, 3)
        scatter(lax.rem(nsc - 1, 2), GB - 1, 3)
        for k in range(4):
            drain(ssem, k)

    def do_tail(table):
        pltpu.sync_copy(tsrc_hbm, srcb.at[0])
        pltpu.sync_copy(tdst_hbm, dstb.at[0])
        for b in range(GB):
            pltpu.async_copy(table.at[srcb.at[0].at[b]], rbuf.at[0],
                             gsem).wait()
            pltpu.async_copy(rbuf.at[0], accum.at[dstb.at[0].at[b]],
                             ssem, add=True).wait()

    @pl.when(c == 0)
    def _():
        do_pass(t0, base0, cnt0)
        if tail0:
            @pl.when(s == 15)
            def _():
                do_tail(t0)

    @pl.when(c == 1)
    def _():
        do_pass(t1, base1, cnt1)
        if tail1:
            @pl.when(s == 15)
            def _():
                do_tail(t1)

    plsc.subcore_barrier()

    @pl.when(c == 0)
    def _():
        pltpu.sync_copy(accum.at[pl.ds(s * ZR, ZR)], out0.at[pl.ds(s * ZR, ZR)])

    @pl.when(c == 1)
    def _():
        pltpu.sync_copy(accum.at[pl.ds(s * ZR, ZR)], out1.at[pl.ds(s * ZR, ZR)])


def _sc_pass(t0, t1, edge3d, tsrc, tdst, zeros, **split):
    f32 = jnp.float32
    body = functools.partial(_sc_scatter_body, **split)
    return pl.kernel(
        body,
        out_type=(jax.ShapeDtypeStruct((NP, 32), f32),
                  jax.ShapeDtypeStruct((NP, 32), f32)),
        mesh=plsc.VectorSubcoreMesh(core_axis_name="c", subcore_axis_name="s"),
        scratch_types=(
            pltpu.VMEM_SHARED((NP, 32), f32),
            pltpu.VMEM((2, GB, 128), jnp.int32),
            pltpu.VMEM((2, GB, 128), jnp.int32),
            pltpu.VMEM((4, 128, 32), f32),
            pltpu.SemaphoreType.DMA,
            pltpu.SemaphoreType.DMA,
            pltpu.SemaphoreType.DMA,
        ),
        compiler_params=pltpu.CompilerParams(use_tc_tiling_on_sc=False),
        name=f"gcn_sc_scatter_{split['cnt0']}",
    )(t0, t1, edge3d, tsrc, tdst, zeros)


def _sc_hist_body(ones_hbm, edge_hbm, tdst_hbm, zeros_hbm, out0, out1,
                  accum, dstb, srcones, isem, ssem):
    c = lax.axis_index("c")
    s = lax.axis_index("s")
    dst_hbm = edge_hbm.at[1]

    pltpu.sync_copy(zeros_hbm.at[pl.ds(s * ZR, ZR)], accum.at[pl.ds(s * ZR, ZR)])
    pltpu.sync_copy(ones_hbm, srcones)
    plsc.subcore_barrier()

    half = NSC_FULL // 2
    base_sc, nsc = _tile_range(half * c, half + c, s)
    base = base_sc * GB

    def load_idx(g, sl):
        pltpu.async_copy(dst_hbm.at[pl.ds(base + g * GB, GB)],
                         dstb.at[sl], isem)

    def drain_scat():
        pltpu.make_async_copy(zeros_hbm.at[pl.ds(0, 128)], srcones,
                              ssem).wait()

    load_idx(0, 0)

    def g_body(g, carry):
        sl = lax.rem(g, 3)
        pltpu.make_async_copy(dst_hbm.at[pl.ds(base, GB)], dstb.at[sl],
                              isem).wait()
        @pl.when(g + 1 < nsc)
        def _():
            load_idx(g + 1, lax.rem(g + 1, 3))
        for b in range(GB):
            @pl.when(g > 0)
            def _():
                drain_scat()
            pltpu.async_copy(srcones, accum.at[dstb.at[sl].at[b]], ssem,
                             add=True)
        return carry

    lax.fori_loop(0, nsc, g_body, 0)
    for _ in range(GB):
        drain_scat()

    @pl.when((c == 1) & (s == 15))
    def _():
        pltpu.sync_copy(tdst_hbm, dstb.at[0])
        for b in range(GB):
            pltpu.async_copy(srcones, accum.at[dstb.at[0].at[b]], ssem,
                             add=True).wait()

    plsc.subcore_barrier()

    @pl.when(c == 0)
    def _():
        pltpu.sync_copy(accum.at[pl.ds(s * ZR, ZR)], out0.at[pl.ds(s * ZR, ZR)])

    @pl.when(c == 1)
    def _():
        pltpu.sync_copy(accum.at[pl.ds(s * ZR, ZR)], out1.at[pl.ds(s * ZR, ZR)])


def _sc_hist(ones32, edge3d, tdst, zeros):
    f32 = jnp.float32
    return pl.kernel(
        _sc_hist_body,
        out_type=(jax.ShapeDtypeStruct((NP, 32), f32),
                  jax.ShapeDtypeStruct((NP, 32), f32)),
        mesh=plsc.VectorSubcoreMesh(core_axis_name="c", subcore_axis_name="s"),
        scratch_types=(
            pltpu.VMEM_SHARED((NP, 32), f32),
            pltpu.VMEM((3, GB, 128), jnp.int32),
            pltpu.VMEM((128, 32), f32),
            pltpu.SemaphoreType.DMA,
            pltpu.SemaphoreType.DMA,
        ),
        compiler_params=pltpu.CompilerParams(use_tc_tiling_on_sc=False),
        name="gcn_sc_hist",
    )(ones32, edge3d, tdst, zeros)


def _t1_body(emb, dega, degb, xs_out, dinv_out):
    deg = dega[...] + degb[...] + 1.0
    dinv = lax.rsqrt(deg)
    dinv_out[...] = dinv
    xs_out[...] = emb[...] * dinv


def _t2_body(p1a, p1b, xs, dinv, Wbd, g1t, b1t, be1t, lo_out, hi_out):
    d = dinv[...]
    P = (p1a[...] + p1b[...] + xs[...]) * d
    z = jnp.dot(P.astype(jnp.bfloat16), Wbd[...],
                preferred_element_type=jnp.float32)
    A = g1t[...] * BN_S
    y = z * A + (b1t[...] * A + be1t[...])
    y = jnp.where(y >= 0, y, ALPHA * y)
    lo_out[...] = y[:, :128] * d
    hi_out[...] = y[:, 128:] * d


def _t3_body(p2lo, p2hi, yslo, yshi, dinv, W2lo, W2hi, g2t, b2t, be2t,
             Wft, bft, out):
    bf16 = jnp.bfloat16
    d = dinv[...]
    Plo = (p2lo[...] + yslo[...]) * d
    Phi = (p2hi[...] + yshi[...]) * d
    z = (jnp.dot(Plo.astype(bf16), W2lo[...], preferred_element_type=jnp.float32)
         + jnp.dot(Phi.astype(bf16), W2hi[...], preferred_element_type=jnp.float32))
    A = g2t[...] * BN_S
    y = z * A + (b2t[...] * A + be2t[...])
    y = jnp.where(y >= 0, y, ALPHA * y)
    o = (jnp.dot(y.astype(bf16), Wft[...], preferred_element_type=jnp.float32)
         + bft[...])
    out[...] = o.reshape(4 * RBLK, 128)


def _row_spec(w):
    return pl.BlockSpec((RBLK, w), lambda i: (i, 0))


def _full_spec(shape):
    nd = len(shape)
    return pl.BlockSpec(shape, lambda i: (0,) * nd)


def _bd4(w):
    return jnp.kron(jnp.eye(4, dtype=w.dtype), w).astype(jnp.bfloat16)


def kernel(embeddings, edge_index, W1, b1, g1, be1, W2, b2, g2, be2, Wf, bf):
    f32 = jnp.float32
    i32 = jnp.int32
    grid = (NQ // RBLK,)

    emb128 = embeddings.reshape(N_NODES * 32 // 128, 128)
    edge3d = edge_index.astype(i32).reshape(2, EROWS, 128)
    trash = N_NODES + jnp.arange(768, dtype=i32) % (NP - N_NODES)
    tail_at = NSC_FULL * GB * 128
    tsrc = jnp.concatenate([edge_index[0, tail_at:].astype(i32),
                            trash]).reshape(GB, 128)
    tdst = jnp.concatenate([edge_index[1, tail_at:].astype(i32),
                            trash]).reshape(GB, 128)
    zeros32 = jnp.zeros((NP, 32), f32)
    ones32 = jnp.ones((128, 32), f32)

    dega, degb = _sc_hist(ones32, edge3d, tdst, zeros32)

    xs, dinv = pl.pallas_call(
        _t1_body,
        grid=grid,
        in_specs=[_row_spec(128), _row_spec(128), _row_spec(128)],
        out_specs=[_row_spec(128), _row_spec(128)],
        out_shape=[jax.ShapeDtypeStruct((NQ, 128), f32),
                   jax.ShapeDtypeStruct((NQ, 128), f32)],
        name="gcn_tc_prep",
    )(emb128, dega.reshape(NQ, 128), degb.reshape(NQ, 128))

    xs32 = xs.reshape(NP, 32)
    half = NSC_FULL // 2
    p1a, p1b = _sc_pass(xs32, xs32, edge3d, tsrc, tdst, zeros32,
                        base0=0, cnt0=half, tail0=False,
                        base1=half, cnt1=half + 1, tail1=True)

    Wbd1 = jnp.concatenate([_bd4(W1[:, :32]), _bd4(W1[:, 32:])], axis=1)
    t4 = lambda v: jnp.tile(v, 4).reshape(1, -1)
    t4s = lambda v: jnp.concatenate([jnp.tile(v[:32], 4),
                                     jnp.tile(v[32:], 4)]).reshape(1, -1)
    ys_lo, ys_hi = pl.pallas_call(
        _t2_body,
        grid=grid,
        in_specs=[_row_spec(128)] * 4 +
                 [_full_spec((128, 256)), _full_spec((1, 256)),
                  _full_spec((1, 256)), _full_spec((1, 256))],
        out_specs=[_row_spec(128), _row_spec(128)],
        out_shape=[jax.ShapeDtypeStruct((NQ, 128), f32),
                   jax.ShapeDtypeStruct((NQ, 128), f32)],
        name="gcn_tc_layer1",
    )(p1a.reshape(NQ, 128), p1b.reshape(NQ, 128), xs, dinv,
      Wbd1, t4s(g1), t4s(b1), t4s(be1))

    p2lo, p2hi = _sc_pass(ys_lo.reshape(NP, 32), ys_hi.reshape(NP, 32),
                          edge3d, tsrc, tdst, zeros32,
                          base0=0, cnt0=NSC_FULL, tail0=True,
                          base1=0, cnt1=NSC_FULL, tail1=True)

    out_w = pl.pallas_call(
        _t3_body,
        grid=grid,
        in_specs=[_row_spec(128)] * 5 +
                 [_full_spec((128, 512)), _full_spec((128, 512)),
                  _full_spec((1, 512)), _full_spec((1, 512)),
                  _full_spec((1, 512)),
                  _full_spec((512, 512)), _full_spec((1, 512))],
        out_specs=pl.BlockSpec((4 * RBLK, 128), lambda i: (i, 0)),
        out_shape=jax.ShapeDtypeStruct((N_NODES, 128), f32),
        name="gcn_tc_layer2_final",
    )(p2lo.reshape(NQ, 128), p2hi.reshape(NQ, 128),
      ys_lo, ys_hi, dinv,
      _bd4(W2[:32, :]), _bd4(W2[32:, :]),
      t4(g2), t4(b2), t4(be2),
      _bd4(Wf), t4(bf))

    return out_w
